# baseline (speedup 1.0000x reference)
import functools

import jax
import jax.numpy as jnp
from jax import lax
from jax.experimental import pallas as pl
from jax.experimental.pallas import tpu as pltpu
from jax.experimental.pallas import tpu_sc as plsc

N = 25000
NE = 800000
D = 32
DW = 16
KM = 128
NW = 32
ROWS = 6400
RW = ROWS // NW
PADN = ROWS * KM - NE
SENT = N
NPAD = N + 24
NB16 = 1564
MAGIC = 21455
MSH = 25
S = 32768
CORE = 16 * S
NBUF = 8
BB = 1000
GB = N // BB


def _sc_mesh():
    return plsc.VectorSubcoreMesh(core_axis_name="c", subcore_axis_name="s")


@functools.lru_cache(maxsize=None)
def _build_bin():
    return functools.partial(
        pl.kernel,
        out_type=(jax.ShapeDtypeStruct((2 * CORE,), jnp.int32),
                  jax.ShapeDtypeStruct((2 * CORE,), jnp.int32),
                  jax.ShapeDtypeStruct((2, 16), jnp.int32)),
        mesh=_sc_mesh(),
        scratch_types=[
            pltpu.VMEM((RW, KM), jnp.int32),
            pltpu.VMEM((RW, KM), jnp.int32),
            pltpu.VMEM((16,), jnp.int32),
            pltpu.VMEM((16, 16), jnp.int32),
            pltpu.VMEM((KM,), jnp.int32),
            pltpu.VMEM((KM,), jnp.int32),
            pltpu.VMEM((KM,), jnp.int32),
            pltpu.VMEM((KM,), jnp.int32),
            pltpu.VMEM((KM,), jnp.int32),
            pltpu.VMEM((KM,), jnp.int32),
            pltpu.VMEM_SHARED((16, 16), jnp.int32),
            pltpu.SemaphoreType.DMA,
            pltpu.SemaphoreType.DMA,
        ],
        compiler_params=pltpu.CompilerParams(use_tc_tiling_on_sc=False,
                                             needs_layout_passes=False),
    )(_bin_body)


def _bin_body(srcg, dstg, esrc, edl, cnt, srcall, dstall, vstage, allc,
              posa, dla, posbB, dlbB, sentb, zerob, shcnt, semA, semB):
    c = lax.axis_index("c")
    s = lax.axis_index("s")
    rowbase = c * (ROWS // 2) + s * RW
    pltpu.sync_copy(srcg.at[pl.ds(rowbase, RW)], srcall)
    pltpu.sync_copy(dstg.at[pl.ds(rowbase, RW)], dstall)
    lane = lax.broadcasted_iota(jnp.int32, (16,), 0)

    def crow(i, cnt16):
        for g in range(8):
            d16 = dstall[i, pl.ds(g * 16, 16)]
            b16 = lax.shift_right_logical(d16 * MAGIC, MSH)
            for l in range(16):
                pc = plsc.all_reduce_population_count(b16 == l)
                cnt16 = cnt16 + jnp.where(lane == l, pc, 0)
        return cnt16

    cnt16 = lax.fori_loop(0, RW, crow, jnp.zeros((16,), jnp.int32))
    vstage[...] = cnt16
    pltpu.sync_copy(vstage, shcnt.at[s])
    plsc.subcore_barrier()
    pltpu.sync_copy(shcnt, allc)

    def accrow(sp, a16):
        return a16 + allc[sp]

    pre16 = lax.fori_loop(0, s, accrow, jnp.zeros((16,), jnp.int32))
    tot16 = lax.fori_loop(0, 16, accrow, jnp.zeros((16,), jnp.int32))
    off0 = c * CORE + lane * S + pre16

    @pl.when(s == 0)
    def _cnt_out():
        vstage[...] = tot16
        pltpu.sync_copy(vstage, cnt.at[c])

    def place_row(i, off16, pb, db, sem):
        for g in range(8):
            d16 = dstall[i, pl.ds(g * 16, 16)]
            b16 = lax.shift_right_logical(d16 * MAGIC, MSH)
            pos = jnp.zeros((16,), jnp.int32)
            for l in range(16):
                m = b16 == l
                rank = plsc.cumsum(m.astype(jnp.int32)) - 1
                off_l = jnp.sum(jnp.where(lane == l, off16, 0))
                pos = jnp.where(m, off_l + rank, pos)
                pc = plsc.all_reduce_population_count(m)
                off16 = off16 + jnp.where(lane == l, pc, 0)
            pb[pl.ds(g * 16, 16)] = pos
            db[pl.ds(g * 16, 16)] = d16 - b16 * NB16
        pltpu.async_copy(srcall.at[i], esrc.at[pb], sem)
        pltpu.async_copy(db, edl.at[pb], sem)
        return off16

    def _drain(sem):
        pltpu.make_async_copy(posa, esrc.at[pl.ds(0, KM)], sem).wait()
        pltpu.make_async_copy(posa, esrc.at[pl.ds(0, KM)], sem).wait()

    def prow2(j, off16):
        @pl.when(j > 0)
        def _():
            _drain(semA)

        off16 = place_row(j * 2, off16, posa, dla, semA)

        @pl.when(j > 0)
        def _():
            _drain(semB)

        off16 = place_row(j * 2 + 1, off16, posbB, dlbB, semB)
        return off16

    lax.fori_loop(0, RW // 2, prow2, off0)
    _drain(semA)
    _drain(semB)

    tot_s = jnp.sum(jnp.where(lane == s, tot16, 0))
    fbase = c * CORE + s * S + tot_s
    for q in range(8):
        sentb[pl.ds(q * 16, 16)] = jnp.full((16,), SENT, jnp.int32)
        zerob[pl.ds(q * 16, 16)] = jnp.zeros((16,), jnp.int32)

    def fill(g, carry):
        for q in range(8):
            posa[pl.ds(q * 16, 16)] = fbase + g * KM + q * 16 + lane
        pltpu.sync_copy(sentb, esrc.at[posa])
        pltpu.sync_copy(zerob, edl.at[posa])
        return carry

    lax.fori_loop(0, NBUF, fill, 0)


@functools.lru_cache(maxsize=None)
def _build_spmm2():
    return functools.partial(
        pl.kernel,
        out_type=jax.ShapeDtypeStruct((2, NPAD, D), jnp.float32),
        mesh=_sc_mesh(),
        scratch_types=(
            [pltpu.VMEM((KM,), jnp.int32) for _ in range(NBUF)] +
            [pltpu.VMEM((KM,), jnp.int32) for _ in range(NBUF)] +
            [pltpu.VMEM((KM, D), jnp.float32) for _ in range(NBUF)] +
            [pltpu.VMEM((NB16, D), jnp.float32),
             pltpu.VMEM((16,), jnp.int32)] +
            [pltpu.SemaphoreType.DMA for _ in range(3 * NBUF)]
        ),
        compiler_params=pltpu.CompilerParams(use_tc_tiling_on_sc=False,
                                             needs_layout_passes=False),
    )(_spmm2_body)


def _spmm2_body(m_hbm, esrc, edl, cnt, out_hbm, *scr):
    idxb = scr[0:NBUF]
    dlb = scr[NBUF:2 * NBUF]
    rowsb = scr[2 * NBUF:3 * NBUF]
    acc = scr[3 * NBUF]
    cntv = scr[3 * NBUF + 1]
    sema = scr[3 * NBUF + 2:3 * NBUF + 2 + NBUF]
    semd = scr[3 * NBUF + 2 + NBUF:3 * NBUF + 2 + 2 * NBUF]
    semg = scr[3 * NBUF + 2 + 2 * NBUF:3 * NBUF + 2 + 3 * NBUF]
    c = lax.axis_index("c")
    s = lax.axis_index("s")
    lane = lax.broadcasted_iota(jnp.int32, (16,), 0)
    pltpu.sync_copy(cnt.at[c], cntv)
    n = jnp.sum(jnp.where(lane == s, cntv[...], 0))
    nbod = lax.shift_right_logical(n + (NBUF * KM - 1), 10)
    base = c * CORE + s * S

    zz = jnp.zeros((16,), jnp.float32)

    def zb(i, carry):
        acc[i, pl.ds(0, 16)] = zz
        acc[i, pl.ds(16, 16)] = zz
        return carry

    lax.fori_loop(0, NB16, zb, 0)

    def start_a(ch, b):
        pltpu.async_copy(esrc.at[pl.ds(base + ch * KM, KM)], idxb[b], sema[b])
        pltpu.async_copy(edl.at[pl.ds(base + ch * KM, KM)], dlb[b], semd[b])

    for b in range(NBUF):
        start_a(b, b)

    def body(j, carry):
        for b in range(NBUF):
            pltpu.make_async_copy(esrc.at[pl.ds(0, KM)], idxb[b], sema[b]).wait()
            pltpu.async_copy(m_hbm.at[idxb[b]], rowsb[b], semg[b])
        for b in range(NBUF):
            pltpu.make_async_copy(esrc.at[pl.ds(0, KM)], dlb[b], semd[b]).wait()
            pltpu.make_async_copy(m_hbm.at[idxb[b]], rowsb[b], semg[b]).wait()

            def gbody(g, c2):
                dl16 = dlb[b][pl.ds(g * 16, 16)]
                e16 = g * 16 + lane
                for j2 in range(D):
                    js = jnp.full((16,), j2, jnp.int32)
                    vals = plsc.load_gather(rowsb[b], [e16, js])
                    plsc.addupdate_scatter(acc, [dl16, js], vals)
                return c2

            lax.fori_loop(0, KM // 16, gbody, 0)
            start_a((j + 1) * NBUF + b, b)
        return carry

    lax.fori_loop(0, nbod, body, 0)
    for b in range(NBUF):
        pltpu.make_async_copy(esrc.at[pl.ds(0, KM)], idxb[b], sema[b]).wait()
        pltpu.make_async_copy(esrc.at[pl.ds(0, KM)], dlb[b], semd[b]).wait()
    pltpu.sync_copy(acc, out_hbm.at[c, pl.ds(s * NB16, NB16)])


@functools.lru_cache(maxsize=None)
def _build_deg():
    return functools.partial(
        pl.kernel,
        out_type=jax.ShapeDtypeStruct((2, NPAD, DW), jnp.float32),
        mesh=_sc_mesh(),
        scratch_types=[
            pltpu.VMEM((RW, KM), jnp.int32),
            pltpu.VMEM((KM, DW), jnp.float32),
            pltpu.VMEM_SHARED((NPAD, DW), jnp.float32),
        ],
        compiler_params=pltpu.CompilerParams(use_tc_tiling_on_sc=False),
    )(_deg_body)


def _deg(srcd, ones_w, zero_w):
    return _build_deg()(srcd, ones_w, zero_w)


def _deg_body(srcd_hbm, ones_hbm, zero_hbm, out_hbm, idx_v, ones_v, acc):
    c = lax.axis_index("c")
    s = lax.axis_index("s")
    wid = s * 2 + c
    base = wid * RW
    pltpu.sync_copy(srcd_hbm.at[pl.ds(base, RW)], idx_v)
    pltpu.sync_copy(ones_hbm, ones_v)

    @pl.when(s == 0)
    def _zero():
        pltpu.sync_copy(zero_hbm, acc)

    plsc.subcore_barrier()

    def body(i, carry):
        pltpu.sync_copy(ones_v, acc.at[idx_v.at[i]], add=True)
        return carry

    lax.fori_loop(0, RW, body, 0)
    plsc.subcore_barrier()

    @pl.when(s == 0)
    def _writeback():
        pltpu.sync_copy(acc, out_hbm.at[c])


def _full(shape):
    return pl.BlockSpec(shape, lambda i: tuple(0 for _ in shape))


def _rows(cols, b=BB):
    return pl.BlockSpec((b, cols), lambda i: (i, 0))


_AGG_SPEC = pl.BlockSpec((2, BB, D), lambda i: (0, i, 0))
_DEG_SPEC = pl.BlockSpec((2, BB, DW), lambda i: (0, i, 0))


def _col_is_last(shape):
    return lax.broadcasted_iota(jnp.int32, shape, 1) == (D - 1)


def _mlp2_body(x_ref, w1_ref, b1_ref, w2_ref, b2_ref, o_ref):
    h = jnp.maximum(x_ref[...] @ w1_ref[...] + b1_ref[...], 0.0)
    o_ref[...] = h @ w2_ref[...] + b2_ref[...]


def _mlp2_tc(x, w1, b1, w2, b2):
    cin = x.shape[1]
    return pl.pallas_call(
        _mlp2_body,
        grid=(GB,),
        in_specs=[_rows(cin), _full(w1.shape), _full(b1.shape),
                  _full(w2.shape), _full(b2.shape)],
        out_specs=_rows(D),
        out_shape=jax.ShapeDtypeStruct((N, D), jnp.float32),
    )(x, w1, b1, w2, b2)


def _v2c_msg_body(x_ref, ef_ref, dg_ref, w1h, b1h, w2h, b2h, w1m, b1m, w2m, b2m, o_ref):
    x = x_ref[...]
    ef = ef_ref[...]
    deg = (dg_ref[0] + dg_ref[1])[:, 0:1]
    norm = jnp.where(deg > 0.5, 1.0 / deg, 0.0)
    t = jax.nn.sigmoid(x @ w1h[...] + b1h[...])
    va = (t @ w2h[...] + b2h[...])[:, 0:1] * ef
    u = jnp.maximum((ef * x) @ w1m[...] + b1m[...], 0.0)
    out = (u @ w2m[...] + b2m[...]) * norm
    o_ref[...] = jnp.where(_col_is_last(out.shape), va, out)


def _v2c_msg(x, ef, dg, wh, wm):
    return pl.pallas_call(
        _v2c_msg_body,
        grid=(GB,),
        in_specs=[_rows(D), _rows(1), _DEG_SPEC] +
                 [_full(w.shape) for w in wh] + [_full(w.shape) for w in wm],
        out_specs=_rows(D),
        out_shape=jax.ShapeDtypeStruct((N, D), jnp.float32),
    )(x, ef, dg, *wh, *wm)


def _v2c_upd_body(ag_ref, old_ref, rhs_ref, wr, br, o_ref):
    aggr = ag_ref[0] + ag_ref[1]
    main = jnp.maximum(aggr + old_ref[...] @ wr[...] + br[...], 0.0)
    last = aggr[:, D - 1:D] - rhs_ref[...]
    o_ref[...] = jnp.where(_col_is_last(main.shape), last, main)


def _v2c_upd(ag, old, rhs2, wr, br):
    return pl.pallas_call(
        _v2c_upd_body,
        grid=(GB,),
        in_specs=[_AGG_SPEC, _rows(D), _rows(1), _full(wr.shape), _full(br.shape)],
        out_specs=_rows(D),
        out_shape=jax.ShapeDtypeStruct((N, D), jnp.float32),
    )(ag, old, rhs2, wr, br)


def _c2v_msg_body(x_ref, ef_ref, dg_ref, w1m, b1m, w2m, b2m, o_ref):
    x = x_ref[...]
    ef = ef_ref[...]
    deg = (dg_ref[0] + dg_ref[1])[:, 0:1]
    norm = jnp.where(deg > 0.5, 1.0 / deg, 0.0)
    u = jnp.maximum((ef * x) @ w1m[...] + b1m[...], 0.0)
    out = u @ w2m[...] + b2m[...]
    bscal = x[:, D - 1:D] * ef
    o_ref[...] = norm * jnp.where(_col_is_last(out.shape), bscal, out)


def _c2v_msg(x, ef, dg, wm):
    return pl.pallas_call(
        _c2v_msg_body,
        grid=(GB,),
        in_specs=[_rows(D), _rows(1), _DEG_SPEC] + [_full(w.shape) for w in wm],
        out_specs=_rows(D),
        out_shape=jax.ShapeDtypeStruct((N, D), jnp.float32),
    )(x, ef, dg, *wm)


def _c2v_upd_body(ag_ref, xd_ref, w1h, b1h, w2h, b2h, wr, br, o_ref):
    aggr = ag_ref[0] + ag_ref[1]
    xd = xd_ref[...]
    t = jax.nn.sigmoid(xd @ w1h[...] + b1h[...])
    a = (t @ w2h[...] + b2h[...])[:, 0:1]
    main = jnp.maximum(aggr + xd @ wr[...] + br[...], 0.0)
    last = a * aggr[:, D - 1:D]
    o_ref[...] = jnp.where(_col_is_last(main.shape), last, main)


def _c2v_upd(ag, xd, wh, wr, br):
    return pl.pallas_call(
        _c2v_upd_body,
        grid=(GB,),
        in_specs=[_AGG_SPEC, _rows(D)] + [_full(w.shape) for w in wh] +
                 [_full(wr.shape), _full(br.shape)],
        out_specs=_rows(D),
        out_shape=jax.ShapeDtypeStruct((N, D), jnp.float32),
    )(ag, xd, *wh, wr, br)


def _head_body(x_ref, *refs):
    o_ref = refs[-1]
    ws = refs[:-1]
    x = x_ref[...]
    for i in range(5):
        x = jnp.maximum(x @ ws[2 * i][...] + ws[2 * i + 1][...], 0.0)
    lg = x @ ws[10][...] + ws[11][...]
    o_ref[...] = jax.nn.log_softmax(lg, axis=1)


def _head(x, ws):
    return pl.pallas_call(
        _head_body,
        grid=(GB,),
        in_specs=[_rows(D)] + [_full(w.shape) for w in ws],
        out_specs=_rows(2),
        out_shape=jax.ShapeDtypeStruct((N, 2), jnp.float32),
    )(x, *ws)


def _pad_mlp2(p, pre):
    w1 = jnp.pad(p[pre + '_W1'], ((0, 0), (0, 1)))
    b1 = jnp.pad(p[pre + '_b1'], (0, 1)).reshape(1, D)
    w2 = jnp.pad(p[pre + '_W2'], ((0, 1), (0, 1)))
    b2 = jnp.pad(p[pre + '_b2'], (0, 1)).reshape(1, D)
    return (w1, b1, w2, b2)


def _pad_h2v(p, l):
    w1 = jnp.pad(p['h2v%d_W1' % l], ((0, 0), (0, 1)))
    b1 = jnp.pad(p['h2v%d_b1' % l], (0, 1)).reshape(1, D)
    w2 = jnp.pad(p['h2v%d_W2' % l], ((0, 1), (0, D - 1)))
    b2 = jnp.pad(p['h2v%d_b2' % l], (0, D - 1)).reshape(1, D)
    return (w1, b1, w2, b2)


def _pad_root(p, pre):
    wr = jnp.pad(p[pre + '_root'], ((0, 0), (0, 1)))
    br = jnp.pad(p[pre + '_bias'], (0, 1)).reshape(1, D)
    return wr, br


_ZROWS = jnp.zeros


def _edge_layouts(src, dst):
    pads = jnp.full((PADN,), SENT, jnp.int32)
    padd = (jnp.arange(PADN, dtype=jnp.int32) * 1627) % NPAD
    g_src = jnp.concatenate([src, pads]).reshape(ROWS, KM)
    g_dst = jnp.concatenate([dst, padd]).reshape(ROWS, KM)
    return g_src, g_dst


def _impl(vnf, cnf, eiv, efv, rhs, eic, efc, asums, params):
    del asums
    p = params
    sv = eiv[0].astype(jnp.int32)
    dv = eiv[1].astype(jnp.int32)
    sc = eic[0].astype(jnp.int32)
    dc = eic[1].astype(jnp.int32)
    svg, dvg = _edge_layouts(sv, dv)
    scg, dcg = _edge_layouts(sc, dc)
    zero_w = jnp.zeros((NPAD, DW), jnp.float32)
    ones_w = jnp.ones((KM, DW), jnp.float32)
    mpad = jnp.zeros((NPAD - N, D), jnp.float32)
    efv = efv.astype(jnp.float32)
    efc = efc.astype(jnp.float32)
    rhs2 = rhs.astype(jnp.float32).reshape(N, 1)

    bin_fn = _build_bin()
    spmm_fn = _build_spmm2()
    esrc_v, edl_v, cnt_v = bin_fn(svg, dvg)
    esrc_c, edl_c, cnt_c = bin_fn(scg, dcg)

    deg_v = _deg(svg, ones_w, zero_w)
    deg_c = _deg(scg, ones_w, zero_w)

    v0 = _mlp2_tc(vnf, *_pad_mlp2(p, 'con_mlp'))
    c0 = _mlp2_tc(cnf, *_pad_mlp2(p, 'var_mlp'))

    x_src, old_cons, vars_ = v0, c0, v0
    for l in (1, 2, 3, 4):
        wh = _pad_h2v(p, l)
        wm_v = _pad_mlp2(p, 'v2c%d' % l)
        mv = _v2c_msg(x_src, efv, deg_v, wh, wm_v)
        ag = spmm_fn(jnp.concatenate([mv, mpad]), esrc_v, edl_v, cnt_v)
        cons = _v2c_upd(ag, old_cons, rhs2, *_pad_root(p, 'v2c%d' % l))

        wm_c = _pad_mlp2(p, 'c2v%d' % l)
        mc = _c2v_msg(cons, efc, deg_c, wm_c)
        ag2 = spmm_fn(jnp.concatenate([mc, mpad]), esrc_c, edl_c, cnt_c)
        wr, br = _pad_root(p, 'c2v%d' % l)
        vars_ = _c2v_upd(ag2, vars_, wh, wr, br)
        x_src, old_cons = vars_, cons

    ws = []
    for i in range(1, 6):
        ws += [p['fc%d_W' % i], p['fc%d_b' % i].reshape(1, D)]
    ws += [p['fc6_W'], p['fc6_b'].reshape(1, 2)]
    return _head(vars_, ws)


def kernel(var_node_features, con_node_features, edge_index_var, edge_features_var, rhs, edge_index_con, edge_features_con, asums, params):
    return _impl(var_node_features, con_node_features, edge_index_var,
                 edge_features_var, rhs, edge_index_con, edge_features_con,
                 asums, params)


# speedup vs baseline: 1.3358x; 1.3358x over previous
import functools

import jax
import jax.numpy as jnp
from jax import lax
from jax.experimental import pallas as pl
from jax.experimental.pallas import tpu as pltpu
from jax.experimental.pallas import tpu_sc as plsc

N = 25000
NE = 800000
D = 32
DW = 8
KM = 128
NW = 32
ROWS = 6400
RW = ROWS // NW
PADN = ROWS * KM - NE
SENT = N
NPAD = N + 24
NB16 = 1564
MAGIC = 21455
MSH = 25
S = 32768
CORE = 16 * S
NBUF = 8
BB = 1000
GB = N // BB


def _sc_mesh():
    return plsc.VectorSubcoreMesh(core_axis_name="c", subcore_axis_name="s")


SENTPACK = SENT << 11


@functools.lru_cache(maxsize=None)
def _build_bin():
    return functools.partial(
        pl.kernel,
        out_type=(jax.ShapeDtypeStruct((2 * CORE,), jnp.int32),
                  jax.ShapeDtypeStruct((2, 16), jnp.int32)),
        mesh=_sc_mesh(),
        scratch_types=[
            pltpu.VMEM((RW, KM), jnp.int32),
            pltpu.VMEM((RW, KM), jnp.int32),
            pltpu.VMEM((16,), jnp.int32),
            pltpu.VMEM((16, 16), jnp.int32),
            pltpu.VMEM((16,), jnp.int32),
            pltpu.VMEM((16,), jnp.int32),
            pltpu.VMEM((17,), jnp.int32),
            pltpu.VMEM((KM,), jnp.int32),
            pltpu.VMEM((KM,), jnp.int32),
            pltpu.VMEM((KM,), jnp.int32),
            pltpu.VMEM((KM,), jnp.int32),
            pltpu.VMEM_SHARED((16, 16), jnp.int32),
            pltpu.SemaphoreType.DMA,
            pltpu.SemaphoreType.DMA,
        ],
        compiler_params=pltpu.CompilerParams(use_tc_tiling_on_sc=False,
                                             needs_layout_passes=False),
    )(_bin_body)


def _bin_body(srcg, dstg, epk, cnt, srcall, dstall, vstage, allc, offarr,
              cntarr, kst, posa, vala, posbB, valbB, shcnt, semA, semB):
    c = lax.axis_index("c")
    s = lax.axis_index("s")
    rowbase = c * (ROWS // 2) + s * RW
    pltpu.sync_copy(srcg.at[pl.ds(rowbase, RW)], srcall)
    pltpu.sync_copy(dstg.at[pl.ds(rowbase, RW)], dstall)
    lane = lax.broadcasted_iota(jnp.int32, (16,), 0)
    ones16 = jnp.ones((16,), jnp.int32)

    cntarr[...] = jnp.zeros((16,), jnp.int32)

    def crow(i, carry):
        for g in range(8):
            d16 = dstall[i, pl.ds(g * 16, 16)]
            b16 = lax.shift_right_logical(d16 * MAGIC, MSH)
            plsc.addupdate_scatter(cntarr, [b16], ones16)
        return carry

    lax.fori_loop(0, RW, crow, 0)
    cnt16 = cntarr[...]
    vstage[...] = cnt16
    pltpu.sync_copy(vstage, shcnt.at[s])
    plsc.subcore_barrier()
    pltpu.sync_copy(shcnt, allc)

    def accrow(sp, a16):
        return a16 + allc[sp]

    pre16 = lax.fori_loop(0, s, accrow, jnp.zeros((16,), jnp.int32))
    tot16 = lax.fori_loop(0, 16, accrow, jnp.zeros((16,), jnp.int32))
    off0 = c * CORE + lane * S + pre16

    @pl.when(s == 0)
    def _cnt_out():
        vstage[...] = tot16
        pltpu.sync_copy(vstage, cnt.at[c])

    offarr[...] = off0
    kst[pl.ds(0, 16)] = jnp.full((16,), -1, jnp.int32)

    def place_row(i, pb, vb, sem):
        isp = i + jnp.zeros((16,), jnp.int32)
        for g in range(8):
            d16 = dstall[i, pl.ds(g * 16, 16)]
            b16 = lax.shift_right_logical(d16 * MAGIC, MSH)
            ks, vs = plsc.sort_key_val(b16, lane)
            kst[pl.ds(1, 16)] = ks
            prev = kst[pl.ds(0, 16)]
            start16 = plsc.cummax(jnp.where(ks != prev, lane, 0))
            rank = lane - start16
            pos = plsc.load_gather(offarr, [ks]) + rank
            plsc.addupdate_scatter(offarr, [ks], ones16)
            col = g * 16 + vs
            s_srt = plsc.load_gather(srcall, [isp, col])
            d_srt = plsc.load_gather(dstall, [isp, col])
            w16 = lax.shift_left(s_srt, 11) | (d_srt - ks * NB16)
            pb[pl.ds(g * 16, 16)] = pos
            vb[pl.ds(g * 16, 16)] = w16
        pltpu.async_copy(vb, epk.at[pb], sem)

    def _drain(sem):
        pltpu.make_async_copy(posa, epk.at[pl.ds(0, KM)], sem).wait()

    def prow2(j, carry):
        @pl.when(j > 0)
        def _():
            _drain(semA)

        place_row(j * 2, posa, vala, semA)

        @pl.when(j > 0)
        def _():
            _drain(semB)

        place_row(j * 2 + 1, posbB, valbB, semB)
        return carry

    lax.fori_loop(0, RW // 2, prow2, 0)
    _drain(semA)
    _drain(semB)

    tot_s = jnp.sum(jnp.where(lane == s, tot16, 0))
    fbase = c * CORE + s * S + tot_s
    for q in range(8):
        vala[pl.ds(q * 16, 16)] = jnp.full((16,), SENTPACK, jnp.int32)

    def fill(g, carry):
        for q in range(8):
            posa[pl.ds(q * 16, 16)] = fbase + g * KM + q * 16 + lane
        pltpu.sync_copy(vala, epk.at[posa])
        return carry

    lax.fori_loop(0, NBUF, fill, 0)


@functools.lru_cache(maxsize=None)
def _build_spmm2():
    return functools.partial(
        pl.kernel,
        out_type=jax.ShapeDtypeStruct((2, NPAD, D), jnp.float32),
        mesh=_sc_mesh(),
        scratch_types=(
            [pltpu.VMEM((KM,), jnp.int32) for _ in range(NBUF)] +
            [pltpu.VMEM((KM,), jnp.int32) for _ in range(NBUF)] +
            [pltpu.VMEM((KM, D), jnp.float32) for _ in range(NBUF)] +
            [pltpu.VMEM((NB16, D), jnp.float32),
             pltpu.VMEM((16,), jnp.int32)] +
            [pltpu.SemaphoreType.DMA for _ in range(2 * NBUF)]
        ),
        compiler_params=pltpu.CompilerParams(use_tc_tiling_on_sc=False,
                                             needs_layout_passes=False),
    )(_spmm2_body)


def _spmm2_body(m_hbm, epk, cnt, out_hbm, *scr):
    pkb = scr[0:NBUF]
    idxb = scr[NBUF:2 * NBUF]
    rowsb = scr[2 * NBUF:3 * NBUF]
    acc = scr[3 * NBUF]
    cntv = scr[3 * NBUF + 1]
    sema = scr[3 * NBUF + 2:3 * NBUF + 2 + NBUF]
    semg = scr[3 * NBUF + 2 + NBUF:3 * NBUF + 2 + 2 * NBUF]
    c = lax.axis_index("c")
    s = lax.axis_index("s")
    lane = lax.broadcasted_iota(jnp.int32, (16,), 0)
    pltpu.sync_copy(cnt.at[c], cntv)
    n = jnp.sum(jnp.where(lane == s, cntv[...], 0))
    nbod = lax.shift_right_logical(n + (NBUF * KM - 1), 10)
    base = c * CORE + s * S

    zz = jnp.zeros((16,), jnp.float32)

    @plsc.parallel_loop(0, NB16, unroll=4)
    def _zero(i):
        acc[i, pl.ds(0, 16)] = zz
        acc[i, pl.ds(16, 16)] = zz

    def start_a(ch, b):
        pltpu.async_copy(epk.at[pl.ds(base + ch * KM, KM)], pkb[b], sema[b])

    for b in range(NBUF):
        start_a(b, b)

    def body(j, carry):
        for b in range(NBUF):
            pltpu.make_async_copy(epk.at[pl.ds(0, KM)], pkb[b], sema[b]).wait()
            for g in range(8):
                w16 = pkb[b][pl.ds(g * 16, 16)]
                idxb[b][pl.ds(g * 16, 16)] = lax.shift_right_logical(w16, 11)
            pltpu.async_copy(m_hbm.at[idxb[b]], rowsb[b], semg[b])
        for b in range(NBUF):
            pltpu.make_async_copy(m_hbm.at[idxb[b]], rowsb[b], semg[b]).wait()

            @plsc.parallel_loop(0, (KM // 16) * D, unroll=4)
            def _accum(i):
                g = lax.shift_right_logical(i, 5)
                j2 = jnp.bitwise_and(i, D - 1)
                dl16 = jnp.bitwise_and(pkb[b][pl.ds(g * 16, 16)], 2047)
                e16 = g * 16 + lane
                js = j2 + jnp.zeros((16,), jnp.int32)
                vals = plsc.load_gather(rowsb[b], [e16, js])
                plsc.addupdate_scatter(acc, [dl16, js], vals)

            start_a((j + 1) * NBUF + b, b)
        return carry

    lax.fori_loop(0, nbod, body, 0)
    for b in range(NBUF):
        pltpu.make_async_copy(epk.at[pl.ds(0, KM)], pkb[b], sema[b]).wait()
    pltpu.sync_copy(acc, out_hbm.at[c, pl.ds(s * NB16, NB16)])


@functools.lru_cache(maxsize=None)
def _build_deg():
    return functools.partial(
        pl.kernel,
        out_type=jax.ShapeDtypeStruct((2, NPAD, DW), jnp.float32),
        mesh=_sc_mesh(),
        scratch_types=[
            pltpu.VMEM((RW, KM), jnp.int32),
            pltpu.VMEM((KM, DW), jnp.float32),
            pltpu.VMEM_SHARED((NPAD, DW), jnp.float32),
        ],
        compiler_params=pltpu.CompilerParams(use_tc_tiling_on_sc=False),
    )(_deg_body)


def _deg(srcd, ones_w, zero_w):
    return _build_deg()(srcd, ones_w, zero_w)


def _deg_body(srcd_hbm, ones_hbm, zero_hbm, out_hbm, idx_v, ones_v, acc):
    c = lax.axis_index("c")
    s = lax.axis_index("s")
    wid = s * 2 + c
    base = wid * RW
    pltpu.sync_copy(srcd_hbm.at[pl.ds(base, RW)], idx_v)
    pltpu.sync_copy(ones_hbm, ones_v)

    @pl.when(s == 0)
    def _zero():
        pltpu.sync_copy(zero_hbm, acc)

    plsc.subcore_barrier()

    def body(i, carry):
        pltpu.sync_copy(ones_v, acc.at[idx_v.at[i]], add=True)
        return carry

    lax.fori_loop(0, RW, body, 0)
    plsc.subcore_barrier()

    @pl.when(s == 0)
    def _writeback():
        pltpu.sync_copy(acc, out_hbm.at[c])


def _full(shape):
    return pl.BlockSpec(shape, lambda i: tuple(0 for _ in shape))


def _rows(cols, b=BB):
    return pl.BlockSpec((b, cols), lambda i: (i, 0))


_AGG_SPEC = pl.BlockSpec((2, BB, D), lambda i: (0, i, 0))
_DEG_SPEC = pl.BlockSpec((2, BB, DW), lambda i: (0, i, 0))


def _col_is_last(shape):
    return lax.broadcasted_iota(jnp.int32, shape, 1) == (D - 1)


def _mlp2_body(x_ref, w1_ref, b1_ref, w2_ref, b2_ref, o_ref):
    h = jnp.maximum(x_ref[...] @ w1_ref[...] + b1_ref[...], 0.0)
    o_ref[...] = h @ w2_ref[...] + b2_ref[...]


def _mlp2_tc(x, w1, b1, w2, b2):
    cin = x.shape[1]
    return pl.pallas_call(
        _mlp2_body,
        grid=(GB,),
        in_specs=[_rows(cin), _full(w1.shape), _full(b1.shape),
                  _full(w2.shape), _full(b2.shape)],
        out_specs=_rows(D),
        out_shape=jax.ShapeDtypeStruct((N, D), jnp.float32),
    )(x, w1, b1, w2, b2)


def _v2c_msg_body(x_ref, ef_ref, dg_ref, w1h, b1h, w2h, b2h, w1m, b1m, w2m, b2m, o_ref):
    x = x_ref[...]
    ef = ef_ref[...]
    deg = (dg_ref[0] + dg_ref[1])[:, 0:1]
    norm = jnp.where(deg > 0.5, 1.0 / deg, 0.0)
    t = jax.nn.sigmoid(x @ w1h[...] + b1h[...])
    va = (t @ w2h[...] + b2h[...])[:, 0:1] * ef
    u = jnp.maximum((ef * x) @ w1m[...] + b1m[...], 0.0)
    out = (u @ w2m[...] + b2m[...]) * norm
    o_ref[...] = jnp.where(_col_is_last(out.shape), va, out)


def _v2c_msg(x, ef, dg, wh, wm):
    return pl.pallas_call(
        _v2c_msg_body,
        grid=(GB,),
        in_specs=[_rows(D), _rows(1), _DEG_SPEC] +
                 [_full(w.shape) for w in wh] + [_full(w.shape) for w in wm],
        out_specs=_rows(D),
        out_shape=jax.ShapeDtypeStruct((N, D), jnp.float32),
    )(x, ef, dg, *wh, *wm)


def _v2c_upd_body(ag_ref, old_ref, rhs_ref, wr, br, o_ref):
    aggr = ag_ref[0] + ag_ref[1]
    main = jnp.maximum(aggr + old_ref[...] @ wr[...] + br[...], 0.0)
    last = aggr[:, D - 1:D] - rhs_ref[...]
    o_ref[...] = jnp.where(_col_is_last(main.shape), last, main)


def _v2c_upd(ag, old, rhs2, wr, br):
    return pl.pallas_call(
        _v2c_upd_body,
        grid=(GB,),
        in_specs=[_AGG_SPEC, _rows(D), _rows(1), _full(wr.shape), _full(br.shape)],
        out_specs=_rows(D),
        out_shape=jax.ShapeDtypeStruct((N, D), jnp.float32),
    )(ag, old, rhs2, wr, br)


def _c2v_msg_body(x_ref, ef_ref, dg_ref, w1m, b1m, w2m, b2m, o_ref):
    x = x_ref[...]
    ef = ef_ref[...]
    deg = (dg_ref[0] + dg_ref[1])[:, 0:1]
    norm = jnp.where(deg > 0.5, 1.0 / deg, 0.0)
    u = jnp.maximum((ef * x) @ w1m[...] + b1m[...], 0.0)
    out = u @ w2m[...] + b2m[...]
    bscal = x[:, D - 1:D] * ef
    o_ref[...] = norm * jnp.where(_col_is_last(out.shape), bscal, out)


def _c2v_msg(x, ef, dg, wm):
    return pl.pallas_call(
        _c2v_msg_body,
        grid=(GB,),
        in_specs=[_rows(D), _rows(1), _DEG_SPEC] + [_full(w.shape) for w in wm],
        out_specs=_rows(D),
        out_shape=jax.ShapeDtypeStruct((N, D), jnp.float32),
    )(x, ef, dg, *wm)


def _c2v_upd_body(ag_ref, xd_ref, w1h, b1h, w2h, b2h, wr, br, o_ref):
    aggr = ag_ref[0] + ag_ref[1]
    xd = xd_ref[...]
    t = jax.nn.sigmoid(xd @ w1h[...] + b1h[...])
    a = (t @ w2h[...] + b2h[...])[:, 0:1]
    main = jnp.maximum(aggr + xd @ wr[...] + br[...], 0.0)
    last = a * aggr[:, D - 1:D]
    o_ref[...] = jnp.where(_col_is_last(main.shape), last, main)


def _c2v_upd(ag, xd, wh, wr, br):
    return pl.pallas_call(
        _c2v_upd_body,
        grid=(GB,),
        in_specs=[_AGG_SPEC, _rows(D)] + [_full(w.shape) for w in wh] +
                 [_full(wr.shape), _full(br.shape)],
        out_specs=_rows(D),
        out_shape=jax.ShapeDtypeStruct((N, D), jnp.float32),
    )(ag, xd, *wh, wr, br)


def _head_body(x_ref, *refs):
    o_ref = refs[-1]
    ws = refs[:-1]
    x = x_ref[...]
    for i in range(5):
        x = jnp.maximum(x @ ws[2 * i][...] + ws[2 * i + 1][...], 0.0)
    lg = x @ ws[10][...] + ws[11][...]
    o_ref[...] = jax.nn.log_softmax(lg, axis=1)


def _head(x, ws):
    return pl.pallas_call(
        _head_body,
        grid=(GB,),
        in_specs=[_rows(D)] + [_full(w.shape) for w in ws],
        out_specs=_rows(2),
        out_shape=jax.ShapeDtypeStruct((N, 2), jnp.float32),
    )(x, *ws)


def _pad_mlp2(p, pre):
    w1 = jnp.pad(p[pre + '_W1'], ((0, 0), (0, 1)))
    b1 = jnp.pad(p[pre + '_b1'], (0, 1)).reshape(1, D)
    w2 = jnp.pad(p[pre + '_W2'], ((0, 1), (0, 1)))
    b2 = jnp.pad(p[pre + '_b2'], (0, 1)).reshape(1, D)
    return (w1, b1, w2, b2)


def _pad_h2v(p, l):
    w1 = jnp.pad(p['h2v%d_W1' % l], ((0, 0), (0, 1)))
    b1 = jnp.pad(p['h2v%d_b1' % l], (0, 1)).reshape(1, D)
    w2 = jnp.pad(p['h2v%d_W2' % l], ((0, 1), (0, D - 1)))
    b2 = jnp.pad(p['h2v%d_b2' % l], (0, D - 1)).reshape(1, D)
    return (w1, b1, w2, b2)


def _pad_root(p, pre):
    wr = jnp.pad(p[pre + '_root'], ((0, 0), (0, 1)))
    br = jnp.pad(p[pre + '_bias'], (0, 1)).reshape(1, D)
    return wr, br


_ZROWS = jnp.zeros


def _edge_layouts(src, dst):
    pads = jnp.full((PADN,), SENT, jnp.int32)
    padd = (jnp.arange(PADN, dtype=jnp.int32) * 1627) % NPAD
    g_src = jnp.concatenate([src, pads]).reshape(ROWS, KM)
    g_dst = jnp.concatenate([dst, padd]).reshape(ROWS, KM)
    return g_src, g_dst


def _impl(vnf, cnf, eiv, efv, rhs, eic, efc, asums, params):
    del asums
    p = params
    sv = eiv[0].astype(jnp.int32)
    dv = eiv[1].astype(jnp.int32)
    sc = eic[0].astype(jnp.int32)
    dc = eic[1].astype(jnp.int32)
    svg, dvg = _edge_layouts(sv, dv)
    scg, dcg = _edge_layouts(sc, dc)
    zero_w = jnp.zeros((NPAD, DW), jnp.float32)
    ones_w = jnp.ones((KM, DW), jnp.float32)
    mpad = jnp.zeros((NPAD - N, D), jnp.float32)
    efv = efv.astype(jnp.float32)
    efc = efc.astype(jnp.float32)
    rhs2 = rhs.astype(jnp.float32).reshape(N, 1)

    bin_fn = _build_bin()
    spmm_fn = _build_spmm2()
    epk_v, cnt_v = bin_fn(svg, dvg)
    epk_c, cnt_c = bin_fn(scg, dcg)

    deg_v = _deg(svg, ones_w, zero_w)
    deg_c = _deg(scg, ones_w, zero_w)

    v0 = _mlp2_tc(vnf, *_pad_mlp2(p, 'con_mlp'))
    c0 = _mlp2_tc(cnf, *_pad_mlp2(p, 'var_mlp'))

    x_src, old_cons, vars_ = v0, c0, v0
    for l in (1, 2, 3, 4):
        wh = _pad_h2v(p, l)
        wm_v = _pad_mlp2(p, 'v2c%d' % l)
        mv = _v2c_msg(x_src, efv, deg_v, wh, wm_v)
        ag = spmm_fn(jnp.concatenate([mv, mpad]), epk_v, cnt_v)
        cons = _v2c_upd(ag, old_cons, rhs2, *_pad_root(p, 'v2c%d' % l))

        wm_c = _pad_mlp2(p, 'c2v%d' % l)
        mc = _c2v_msg(cons, efc, deg_c, wm_c)
        ag2 = spmm_fn(jnp.concatenate([mc, mpad]), epk_c, cnt_c)
        wr, br = _pad_root(p, 'c2v%d' % l)
        vars_ = _c2v_upd(ag2, vars_, wh, wr, br)
        x_src, old_cons = vars_, cons

    ws = []
    for i in range(1, 6):
        ws += [p['fc%d_W' % i], p['fc%d_b' % i].reshape(1, D)]
    ws += [p['fc6_W'], p['fc6_b'].reshape(1, 2)]
    return _head(vars_, ws)


def kernel(var_node_features, con_node_features, edge_index_var, edge_features_var, rhs, edge_index_con, edge_features_con, asums, params):
    return _impl(var_node_features, con_node_features, edge_index_var,
                 edge_features_var, rhs, edge_index_con, edge_features_con,
                 asums, params)


# speedup vs baseline: 1.7107x; 1.2807x over previous
import functools

import jax
import jax.numpy as jnp
from jax import lax
from jax.experimental import pallas as pl
from jax.experimental.pallas import tpu as pltpu
from jax.experimental.pallas import tpu_sc as plsc

N = 25000
NE = 800000
D = 32
DW = 8
KM = 128
NW = 32
ROWS = 6400
RW = ROWS // NW
PADN = ROWS * KM - NE
SENT = N
NPAD = N + 24
NB16 = 1564
MAGIC = 21455
MSH = 25
S = 32768
CORE = 16 * S
NBUF = 8
D33 = 33
BB = 1000
GB = N // BB


def _sc_mesh():
    return plsc.VectorSubcoreMesh(core_axis_name="c", subcore_axis_name="s")


SENTPACK = SENT << 11


@functools.lru_cache(maxsize=None)
def _build_bin():
    return functools.partial(
        pl.kernel,
        out_type=(jax.ShapeDtypeStruct((2 * CORE,), jnp.int32),
                  jax.ShapeDtypeStruct((2, 16), jnp.int32)),
        mesh=_sc_mesh(),
        scratch_types=[
            pltpu.VMEM((RW, KM), jnp.int32),
            pltpu.VMEM((RW, KM), jnp.int32),
            pltpu.VMEM((16,), jnp.int32),
            pltpu.VMEM((16, 16), jnp.int32),
            pltpu.VMEM((64,), jnp.int32),
            pltpu.VMEM((64,), jnp.int32),
            pltpu.VMEM((16,), jnp.int32),
            pltpu.VMEM((4, 17), jnp.int32),
            pltpu.VMEM((KM,), jnp.int32),
            pltpu.VMEM((KM,), jnp.int32),
            pltpu.VMEM((KM,), jnp.int32),
            pltpu.VMEM((KM,), jnp.int32),
            pltpu.VMEM_SHARED((16, 16), jnp.int32),
            pltpu.SemaphoreType.DMA,
            pltpu.SemaphoreType.DMA,
        ],
        compiler_params=pltpu.CompilerParams(use_tc_tiling_on_sc=False,
                                             needs_layout_passes=False),
    )(_bin_body)


def _bin_body(srcg, dstg, epk, cnt, srcall, dstall, vstage, allc, gc, bases,
              cntarr, kst, posa, vala, posbB, valbB, shcnt, semA, semB):
    c = lax.axis_index("c")
    s = lax.axis_index("s")
    rowbase = c * (ROWS // 2) + s * RW
    pltpu.sync_copy(srcg.at[pl.ds(rowbase, RW)], srcall)
    pltpu.sync_copy(dstg.at[pl.ds(rowbase, RW)], dstall)
    lane = lax.broadcasted_iota(jnp.int32, (16,), 0)
    ones16 = jnp.ones((16,), jnp.int32)

    cntarr[...] = jnp.zeros((16,), jnp.int32)

    def crow(i, carry):
        for g in range(8):
            d16 = dstall[i, pl.ds(g * 16, 16)]
            b16 = lax.shift_right_logical(d16 * MAGIC, MSH)
            plsc.addupdate_scatter(cntarr, [b16], ones16)
        return carry

    lax.fori_loop(0, RW, crow, 0)
    cnt16 = cntarr[...]
    vstage[...] = cnt16
    pltpu.sync_copy(vstage, shcnt.at[s])
    plsc.subcore_barrier()
    pltpu.sync_copy(shcnt, allc)

    def accrow(sp, a16):
        return a16 + allc[sp]

    pre16 = lax.fori_loop(0, s, accrow, jnp.zeros((16,), jnp.int32))
    tot16 = lax.fori_loop(0, 16, accrow, jnp.zeros((16,), jnp.int32))
    off0 = c * CORE + lane * S + pre16

    @pl.when(s == 0)
    def _cnt_out():
        vstage[...] = tot16
        pltpu.sync_copy(vstage, cnt.at[c])

    for q in range(4):
        kst[q, pl.ds(0, 16)] = jnp.full((16,), -1, jnp.int32)
    zero16 = jnp.zeros((16,), jnp.int32)

    def place_half(i, g0, off16, pb, vb):
        isp = i + zero16
        b16s = []
        for q in range(4):
            g = g0 + q
            d16 = dstall[i, pl.ds(g * 16, 16)]
            b16 = lax.shift_right_logical(d16 * MAGIC, MSH)
            b16s.append(b16)
            gc[pl.ds(q * 16, 16)] = zero16
            plsc.addupdate_scatter(gc, [q * 16 + b16], ones16)
        base_q = off16
        for q in range(4):
            bases[pl.ds(q * 16, 16)] = base_q
            base_q = base_q + gc[pl.ds(q * 16, 16)]
        for q in range(4):
            g = g0 + q
            b16 = b16s[q]
            ks, vs = plsc.sort_key_val(b16, lane)
            kst[q, pl.ds(1, 16)] = ks
            prev = kst[q, pl.ds(0, 16)]
            start16 = plsc.cummax(jnp.where(ks != prev, lane, 0))
            rank = lane - start16
            pos = plsc.load_gather(bases, [q * 16 + ks]) + rank
            col = g * 16 + vs
            s_srt = plsc.load_gather(srcall, [isp, col])
            d_srt = plsc.load_gather(dstall, [isp, col])
            w16 = lax.shift_left(s_srt, 11) | (d_srt - ks * NB16)
            pb[pl.ds(g * 16, 16)] = pos
            vb[pl.ds(g * 16, 16)] = w16
        return base_q

    def place_row(i, off16, pb, vb, sem):
        off16 = place_half(i, 0, off16, pb, vb)
        off16 = place_half(i, 4, off16, pb, vb)
        pltpu.async_copy(vb, epk.at[pb], sem)
        return off16

    def _drain(sem):
        pltpu.make_async_copy(posa, epk.at[pl.ds(0, KM)], sem).wait()

    def prow2(j, off16):
        @pl.when(j > 0)
        def _():
            _drain(semA)

        off16 = place_row(j * 2, off16, posa, vala, semA)

        @pl.when(j > 0)
        def _():
            _drain(semB)

        off16 = place_row(j * 2 + 1, off16, posbB, valbB, semB)
        return off16

    lax.fori_loop(0, RW // 2, prow2, off0)
    _drain(semA)
    _drain(semB)

    tot_s = jnp.sum(jnp.where(lane == s, tot16, 0))
    fbase = c * CORE + s * S + tot_s
    for q in range(8):
        vala[pl.ds(q * 16, 16)] = jnp.full((16,), SENTPACK, jnp.int32)

    def fill(g, carry):
        for q in range(8):
            posa[pl.ds(q * 16, 16)] = fbase + g * KM + q * 16 + lane
        pltpu.sync_copy(vala, epk.at[posa])
        return carry

    lax.fori_loop(0, NBUF, fill, 0)


@functools.lru_cache(maxsize=None)
def _build_spmm2():
    return functools.partial(
        pl.kernel,
        out_type=jax.ShapeDtypeStruct((2, NPAD, D33), jnp.float32),
        mesh=_sc_mesh(),
        scratch_types=(
            [pltpu.VMEM((KM,), jnp.int32) for _ in range(NBUF)] +
            [pltpu.VMEM((KM,), jnp.int32) for _ in range(NBUF)] +
            [pltpu.VMEM((KM, D33), jnp.float32) for _ in range(NBUF)] +
            [pltpu.VMEM((NB16, D33), jnp.float32),
             pltpu.VMEM((16,), jnp.int32)] +
            [pltpu.SemaphoreType.DMA for _ in range(2 * NBUF)]
        ),
        compiler_params=pltpu.CompilerParams(use_tc_tiling_on_sc=False,
                                             needs_layout_passes=False),
    )(_spmm2_body)


def _spmm2_body(m_hbm, epk, cnt, out_hbm, *scr):
    pkb = scr[0:NBUF]
    idxb = scr[NBUF:2 * NBUF]
    rowsb = scr[2 * NBUF:3 * NBUF]
    acc = scr[3 * NBUF]
    cntv = scr[3 * NBUF + 1]
    sema = scr[3 * NBUF + 2:3 * NBUF + 2 + NBUF]
    semg = scr[3 * NBUF + 2 + NBUF:3 * NBUF + 2 + 2 * NBUF]
    c = lax.axis_index("c")
    s = lax.axis_index("s")
    lane = lax.broadcasted_iota(jnp.int32, (16,), 0)
    pltpu.sync_copy(cnt.at[c], cntv)
    n = jnp.sum(jnp.where(lane == s, cntv[...], 0))
    nbod = lax.shift_right_logical(n + (NBUF * KM - 1), 10)
    base = c * CORE + s * S

    zz = jnp.zeros((16,), jnp.float32)

    @plsc.parallel_loop(0, NB16, unroll=4)
    def _zero(i):
        acc[i, pl.ds(0, 16)] = zz
        acc[i, pl.ds(16, 16)] = zz

    def start_a(ch, b):
        pltpu.async_copy(epk.at[pl.ds(base + ch * KM, KM)], pkb[b], sema[b])

    for b in range(NBUF):
        start_a(b, b)

    def body(j, carry):
        for b in range(NBUF):
            pltpu.make_async_copy(epk.at[pl.ds(0, KM)], pkb[b], sema[b]).wait()
            for g in range(8):
                w16 = pkb[b][pl.ds(g * 16, 16)]
                idxb[b][pl.ds(g * 16, 16)] = lax.shift_right_logical(w16, 11)
            pltpu.async_copy(m_hbm.at[idxb[b]], rowsb[b], semg[b])
        for b in range(NBUF):
            pltpu.make_async_copy(m_hbm.at[idxb[b]], rowsb[b], semg[b]).wait()

            @plsc.parallel_loop(0, (KM // 16) * D, unroll=4)
            def _accum(i):
                g = lax.shift_right_logical(i, 5)
                j2 = jnp.bitwise_and(i, D - 1)
                dl16 = jnp.bitwise_and(pkb[b][pl.ds(g * 16, 16)], 2047)
                e16 = g * 16 + lane
                js = j2 + jnp.zeros((16,), jnp.int32)
                vals = plsc.load_gather(rowsb[b], [e16, js])
                plsc.addupdate_scatter(acc, [dl16, js], vals)

            start_a((j + 1) * NBUF + b, b)
        return carry

    lax.fori_loop(0, nbod, body, 0)
    for b in range(NBUF):
        pltpu.make_async_copy(epk.at[pl.ds(0, KM)], pkb[b], sema[b]).wait()
    pltpu.sync_copy(acc, out_hbm.at[c, pl.ds(s * NB16, NB16)])


@functools.lru_cache(maxsize=None)
def _build_deg():
    return functools.partial(
        pl.kernel,
        out_type=jax.ShapeDtypeStruct((2, NPAD, DW), jnp.float32),
        mesh=_sc_mesh(),
        scratch_types=[
            pltpu.VMEM((RW, KM), jnp.int32),
            pltpu.VMEM((KM, DW), jnp.float32),
            pltpu.VMEM_SHARED((NPAD, DW), jnp.float32),
        ],
        compiler_params=pltpu.CompilerParams(use_tc_tiling_on_sc=False),
    )(_deg_body)


def _deg(srcd, ones_w, zero_w):
    return _build_deg()(srcd, ones_w, zero_w)


def _deg_body(srcd_hbm, ones_hbm, zero_hbm, out_hbm, idx_v, ones_v, acc):
    c = lax.axis_index("c")
    s = lax.axis_index("s")
    wid = s * 2 + c
    base = wid * RW
    pltpu.sync_copy(srcd_hbm.at[pl.ds(base, RW)], idx_v)
    pltpu.sync_copy(ones_hbm, ones_v)

    @pl.when(s == 0)
    def _zero():
        pltpu.sync_copy(zero_hbm, acc)

    plsc.subcore_barrier()

    def body(i, carry):
        pltpu.sync_copy(ones_v, acc.at[idx_v.at[i]], add=True)
        return carry

    lax.fori_loop(0, RW, body, 0)
    plsc.subcore_barrier()

    @pl.when(s == 0)
    def _writeback():
        pltpu.sync_copy(acc, out_hbm.at[c])


def _full(shape):
    return pl.BlockSpec(shape, lambda i: tuple(0 for _ in shape))


def _rows(cols, b=BB):
    return pl.BlockSpec((b, cols), lambda i: (i, 0))


_AGG_SPEC = pl.BlockSpec((2, BB, D33), lambda i: (0, i, 0))
_DEG_SPEC = pl.BlockSpec((2, BB, DW), lambda i: (0, i, 0))


def _col_is_last(shape):
    return lax.broadcasted_iota(jnp.int32, shape, 1) == (D - 1)


def _mlp2_body(x_ref, w1_ref, b1_ref, w2_ref, b2_ref, o_ref):
    h = jnp.maximum(x_ref[...] @ w1_ref[...] + b1_ref[...], 0.0)
    o_ref[...] = h @ w2_ref[...] + b2_ref[...]


def _mlp2_tc(x, w1, b1, w2, b2):
    cin = x.shape[1]
    return pl.pallas_call(
        _mlp2_body,
        grid=(GB,),
        in_specs=[_rows(cin), _full(w1.shape), _full(b1.shape),
                  _full(w2.shape), _full(b2.shape)],
        out_specs=_rows(D),
        out_shape=jax.ShapeDtypeStruct((N, D), jnp.float32),
    )(x, w1, b1, w2, b2)


def _v2c_msg_body(x_ref, ef_ref, dg_ref, w1h, b1h, w2h, b2h, w1m, b1m, w2m, b2m, o_ref):
    x = x_ref[...]
    ef = ef_ref[...]
    deg = (dg_ref[0] + dg_ref[1])[:, 0:1]
    norm = jnp.where(deg > 0.5, 1.0 / deg, 0.0)
    t = jax.nn.sigmoid(x @ w1h[...] + b1h[...])
    va = (t @ w2h[...] + b2h[...])[:, 0:1] * ef
    u = jnp.maximum((ef * x) @ w1m[...] + b1m[...], 0.0)
    out = (u @ w2m[...] + b2m[...]) * norm
    o_ref[...] = jnp.where(_col_is_last(out.shape), va, out)


def _v2c_msg(x, ef, dg, wh, wm):
    return pl.pallas_call(
        _v2c_msg_body,
        grid=(GB,),
        in_specs=[_rows(D), _rows(1), _DEG_SPEC] +
                 [_full(w.shape) for w in wh] + [_full(w.shape) for w in wm],
        out_specs=_rows(D),
        out_shape=jax.ShapeDtypeStruct((N, D), jnp.float32),
    )(x, ef, dg, *wh, *wm)


def _v2c_upd_body(ag_ref, old_ref, rhs_ref, wr, br, o_ref):
    aggr = (ag_ref[0] + ag_ref[1])[:, :D]
    main = jnp.maximum(aggr + old_ref[...] @ wr[...] + br[...], 0.0)
    last = aggr[:, D - 1:D] - rhs_ref[...]
    o_ref[...] = jnp.where(_col_is_last(main.shape), last, main)


def _v2c_upd(ag, old, rhs2, wr, br):
    return pl.pallas_call(
        _v2c_upd_body,
        grid=(GB,),
        in_specs=[_AGG_SPEC, _rows(D), _rows(1), _full(wr.shape), _full(br.shape)],
        out_specs=_rows(D),
        out_shape=jax.ShapeDtypeStruct((N, D), jnp.float32),
    )(ag, old, rhs2, wr, br)


def _c2v_msg_body(x_ref, ef_ref, dg_ref, w1m, b1m, w2m, b2m, o_ref):
    x = x_ref[...]
    ef = ef_ref[...]
    deg = (dg_ref[0] + dg_ref[1])[:, 0:1]
    norm = jnp.where(deg > 0.5, 1.0 / deg, 0.0)
    u = jnp.maximum((ef * x) @ w1m[...] + b1m[...], 0.0)
    out = u @ w2m[...] + b2m[...]
    bscal = x[:, D - 1:D] * ef
    o_ref[...] = norm * jnp.where(_col_is_last(out.shape), bscal, out)


def _c2v_msg(x, ef, dg, wm):
    return pl.pallas_call(
        _c2v_msg_body,
        grid=(GB,),
        in_specs=[_rows(D), _rows(1), _DEG_SPEC] + [_full(w.shape) for w in wm],
        out_specs=_rows(D),
        out_shape=jax.ShapeDtypeStruct((N, D), jnp.float32),
    )(x, ef, dg, *wm)


def _c2v_upd_body(ag_ref, xd_ref, w1h, b1h, w2h, b2h, wr, br, o_ref):
    aggr = (ag_ref[0] + ag_ref[1])[:, :D]
    xd = xd_ref[...]
    t = jax.nn.sigmoid(xd @ w1h[...] + b1h[...])
    a = (t @ w2h[...] + b2h[...])[:, 0:1]
    main = jnp.maximum(aggr + xd @ wr[...] + br[...], 0.0)
    last = a * aggr[:, D - 1:D]
    o_ref[...] = jnp.where(_col_is_last(main.shape), last, main)


def _c2v_upd(ag, xd, wh, wr, br):
    return pl.pallas_call(
        _c2v_upd_body,
        grid=(GB,),
        in_specs=[_AGG_SPEC, _rows(D)] + [_full(w.shape) for w in wh] +
                 [_full(wr.shape), _full(br.shape)],
        out_specs=_rows(D),
        out_shape=jax.ShapeDtypeStruct((N, D), jnp.float32),
    )(ag, xd, *wh, wr, br)


def _head_body(x_ref, *refs):
    o_ref = refs[-1]
    ws = refs[:-1]
    x = x_ref[...]
    for i in range(5):
        x = jnp.maximum(x @ ws[2 * i][...] + ws[2 * i + 1][...], 0.0)
    lg = x @ ws[10][...] + ws[11][...]
    o_ref[...] = jax.nn.log_softmax(lg, axis=1)


def _head(x, ws):
    return pl.pallas_call(
        _head_body,
        grid=(GB,),
        in_specs=[_rows(D)] + [_full(w.shape) for w in ws],
        out_specs=_rows(2),
        out_shape=jax.ShapeDtypeStruct((N, 2), jnp.float32),
    )(x, *ws)


def _pad_mlp2(p, pre):
    w1 = jnp.pad(p[pre + '_W1'], ((0, 0), (0, 1)))
    b1 = jnp.pad(p[pre + '_b1'], (0, 1)).reshape(1, D)
    w2 = jnp.pad(p[pre + '_W2'], ((0, 1), (0, 1)))
    b2 = jnp.pad(p[pre + '_b2'], (0, 1)).reshape(1, D)
    return (w1, b1, w2, b2)


def _pad_h2v(p, l):
    w1 = jnp.pad(p['h2v%d_W1' % l], ((0, 0), (0, 1)))
    b1 = jnp.pad(p['h2v%d_b1' % l], (0, 1)).reshape(1, D)
    w2 = jnp.pad(p['h2v%d_W2' % l], ((0, 1), (0, D - 1)))
    b2 = jnp.pad(p['h2v%d_b2' % l], (0, D - 1)).reshape(1, D)
    return (w1, b1, w2, b2)


def _pad_root(p, pre):
    wr = jnp.pad(p[pre + '_root'], ((0, 0), (0, 1)))
    br = jnp.pad(p[pre + '_bias'], (0, 1)).reshape(1, D)
    return wr, br


_ZROWS = jnp.zeros


def _edge_layouts(src, dst):
    pads = jnp.full((PADN,), SENT, jnp.int32)
    padd = (jnp.arange(PADN, dtype=jnp.int32) * 1627) % NPAD
    g_src = jnp.concatenate([src, pads]).reshape(ROWS, KM)
    g_dst = jnp.concatenate([dst, padd]).reshape(ROWS, KM)
    return g_src, g_dst


def _impl(vnf, cnf, eiv, efv, rhs, eic, efc, asums, params):
    del asums
    p = params
    sv = eiv[0].astype(jnp.int32)
    dv = eiv[1].astype(jnp.int32)
    sc = eic[0].astype(jnp.int32)
    dc = eic[1].astype(jnp.int32)
    svg, dvg = _edge_layouts(sv, dv)
    scg, dcg = _edge_layouts(sc, dc)
    zero_w = jnp.zeros((NPAD, DW), jnp.float32)
    ones_w = jnp.ones((KM, DW), jnp.float32)
    efv = efv.astype(jnp.float32)
    efc = efc.astype(jnp.float32)
    rhs2 = rhs.astype(jnp.float32).reshape(N, 1)

    bin_fn = _build_bin()
    spmm_fn = _build_spmm2()
    epk_v, cnt_v = bin_fn(svg, dvg)
    epk_c, cnt_c = bin_fn(scg, dcg)

    deg_v = _deg(svg, ones_w, zero_w)
    deg_c = _deg(scg, ones_w, zero_w)

    v0 = _mlp2_tc(vnf, *_pad_mlp2(p, 'con_mlp'))
    c0 = _mlp2_tc(cnf, *_pad_mlp2(p, 'var_mlp'))

    x_src, old_cons, vars_ = v0, c0, v0
    for l in (1, 2, 3, 4):
        wh = _pad_h2v(p, l)
        wm_v = _pad_mlp2(p, 'v2c%d' % l)
        mv = _v2c_msg(x_src, efv, deg_v, wh, wm_v)
        ag = spmm_fn(jnp.pad(mv, ((0, NPAD - N), (0, 1))), epk_v, cnt_v)
        cons = _v2c_upd(ag, old_cons, rhs2, *_pad_root(p, 'v2c%d' % l))

        wm_c = _pad_mlp2(p, 'c2v%d' % l)
        mc = _c2v_msg(cons, efc, deg_c, wm_c)
        ag2 = spmm_fn(jnp.pad(mc, ((0, NPAD - N), (0, 1))), epk_c, cnt_c)
        wr, br = _pad_root(p, 'c2v%d' % l)
        vars_ = _c2v_upd(ag2, vars_, wh, wr, br)
        x_src, old_cons = vars_, cons

    ws = []
    for i in range(1, 6):
        ws += [p['fc%d_W' % i], p['fc%d_b' % i].reshape(1, D)]
    ws += [p['fc6_W'], p['fc6_b'].reshape(1, 2)]
    return _head(vars_, ws)


def kernel(var_node_features, con_node_features, edge_index_var, edge_features_var, rhs, edge_index_con, edge_features_con, asums, params):
    return _impl(var_node_features, con_node_features, edge_index_var,
                 edge_features_var, rhs, edge_index_con, edge_features_con,
                 asums, params)


# speedup vs baseline: 3.8620x; 2.2575x over previous
import functools

import jax
import jax.numpy as jnp
from jax import lax
from jax.experimental import pallas as pl
from jax.experimental.pallas import tpu as pltpu
from jax.experimental.pallas import tpu_sc as plsc

N = 25000
NE = 800000
D = 32
DW = 8
KM = 128
NW = 32
ROWS = 6400
RW = ROWS // NW
PADN = ROWS * KM - NE
SENT = N
NPAD = N + 8
BB = 1000
GB = N // BB


@functools.lru_cache(maxsize=None)
def _build_spmm():
    mesh = plsc.VectorSubcoreMesh(core_axis_name="c", subcore_axis_name="s")
    return functools.partial(
        pl.kernel,
        out_type=jax.ShapeDtypeStruct((2, NPAD, D), jnp.float32),
        mesh=mesh,
        scratch_types=[
            pltpu.VMEM((RW, KM), jnp.int32),
            pltpu.VMEM((RW, KM), jnp.int32),
            pltpu.VMEM((KM, D), jnp.float32),
            pltpu.VMEM_SHARED((NPAD, D), jnp.float32),
            pltpu.SemaphoreType.DMA,
        ],
        compiler_params=pltpu.CompilerParams(use_tc_tiling_on_sc=False),
    )(_spmm_body)


def _spmm(m, srcg, dstg, zero):
    return _build_spmm()(m, srcg, dstg, zero)


def _spmm_body(m_hbm, srcg_hbm, dstg_hbm, zero_hbm, out_hbm, idx_v, didx_v, rows_v, acc, sem):
    c = lax.axis_index("c")
    s = lax.axis_index("s")
    wid = s * 2 + c
    base = wid * RW
    pltpu.sync_copy(srcg_hbm.at[pl.ds(base, RW)], idx_v)
    pltpu.sync_copy(dstg_hbm.at[pl.ds(base, RW)], didx_v)

    @pl.when(s == 0)
    def _zero():
        pltpu.sync_copy(zero_hbm, acc)

    plsc.subcore_barrier()

    def body(i, carry):
        pltpu.async_copy(m_hbm.at[idx_v.at[i]], rows_v, sem).wait()
        pltpu.sync_copy(rows_v, acc.at[didx_v.at[i]], add=True)
        return carry

    lax.fori_loop(0, RW, body, 0)
    plsc.subcore_barrier()

    @pl.when(s == 0)
    def _writeback():
        pltpu.sync_copy(acc, out_hbm.at[c])


@functools.lru_cache(maxsize=None)
def _build_deg():
    mesh = plsc.VectorSubcoreMesh(core_axis_name="c", subcore_axis_name="s")
    return functools.partial(
        pl.kernel,
        out_type=jax.ShapeDtypeStruct((2, NPAD, DW), jnp.float32),
        mesh=mesh,
        scratch_types=[
            pltpu.VMEM((RW, KM), jnp.int32),
            pltpu.VMEM((KM, DW), jnp.float32),
            pltpu.VMEM_SHARED((NPAD, DW), jnp.float32),
        ],
        compiler_params=pltpu.CompilerParams(use_tc_tiling_on_sc=False),
    )(_deg_body)


def _deg(srcd, ones_w, zero_w):
    return _build_deg()(srcd, ones_w, zero_w)


def _deg_body(srcd_hbm, ones_hbm, zero_hbm, out_hbm, idx_v, ones_v, acc):
    c = lax.axis_index("c")
    s = lax.axis_index("s")
    wid = s * 2 + c
    base = wid * RW
    pltpu.sync_copy(srcd_hbm.at[pl.ds(base, RW)], idx_v)
    pltpu.sync_copy(ones_hbm, ones_v)

    @pl.when(s == 0)
    def _zero():
        pltpu.sync_copy(zero_hbm, acc)

    plsc.subcore_barrier()

    def body(i, carry):
        pltpu.sync_copy(ones_v, acc.at[idx_v.at[i]], add=True)
        return carry

    lax.fori_loop(0, RW, body, 0)
    plsc.subcore_barrier()

    @pl.when(s == 0)
    def _writeback():
        pltpu.sync_copy(acc, out_hbm.at[c])


def _full(shape):
    return pl.BlockSpec(shape, lambda i: tuple(0 for _ in shape))


def _rows(cols, b=BB):
    return pl.BlockSpec((b, cols), lambda i: (i, 0))


_AGG_SPEC = pl.BlockSpec((2, BB, D), lambda i: (0, i, 0))
_DEG_SPEC = pl.BlockSpec((2, BB, DW), lambda i: (0, i, 0))


def _col_is_last(shape):
    return lax.broadcasted_iota(jnp.int32, shape, 1) == (D - 1)


def _mlp2_body(x_ref, w1_ref, b1_ref, w2_ref, b2_ref, o_ref):
    h = jnp.maximum(x_ref[...] @ w1_ref[...] + b1_ref[...], 0.0)
    o_ref[...] = h @ w2_ref[...] + b2_ref[...]


def _mlp2_tc(x, w1, b1, w2, b2):
    cin = x.shape[1]
    return pl.pallas_call(
        _mlp2_body,
        grid=(GB,),
        in_specs=[_rows(cin), _full(w1.shape), _full(b1.shape),
                  _full(w2.shape), _full(b2.shape)],
        out_specs=_rows(D),
        out_shape=jax.ShapeDtypeStruct((N, D), jnp.float32),
    )(x, w1, b1, w2, b2)


def _v2c_msg_body(x_ref, ef_ref, dg_ref, w1h, b1h, w2h, b2h, w1m, b1m, w2m, b2m, o_ref):
    x = x_ref[...]
    ef = ef_ref[...]
    deg = (dg_ref[0] + dg_ref[1])[:, 0:1]
    norm = jnp.where(deg > 0.5, 1.0 / deg, 0.0)
    t = jax.nn.sigmoid(x @ w1h[...] + b1h[...])
    va = (t @ w2h[...] + b2h[...])[:, 0:1] * ef
    u = jnp.maximum((ef * x) @ w1m[...] + b1m[...], 0.0)
    out = (u @ w2m[...] + b2m[...]) * norm
    o_ref[...] = jnp.where(_col_is_last(out.shape), va, out)


def _v2c_msg(x, ef, dg, wh, wm):
    return pl.pallas_call(
        _v2c_msg_body,
        grid=(GB,),
        in_specs=[_rows(D), _rows(1), _DEG_SPEC] +
                 [_full(w.shape) for w in wh] + [_full(w.shape) for w in wm],
        out_specs=_rows(D),
        out_shape=jax.ShapeDtypeStruct((N, D), jnp.float32),
    )(x, ef, dg, *wh, *wm)


def _v2c_upd_body(ag_ref, old_ref, rhs_ref, wr, br, o_ref):
    aggr = ag_ref[0] + ag_ref[1]
    main = jnp.maximum(aggr + old_ref[...] @ wr[...] + br[...], 0.0)
    last = aggr[:, D - 1:D] - rhs_ref[...]
    o_ref[...] = jnp.where(_col_is_last(main.shape), last, main)


def _v2c_upd(ag, old, rhs2, wr, br):
    return pl.pallas_call(
        _v2c_upd_body,
        grid=(GB,),
        in_specs=[_AGG_SPEC, _rows(D), _rows(1), _full(wr.shape), _full(br.shape)],
        out_specs=_rows(D),
        out_shape=jax.ShapeDtypeStruct((N, D), jnp.float32),
    )(ag, old, rhs2, wr, br)


def _c2v_msg_body(x_ref, ef_ref, dg_ref, w1m, b1m, w2m, b2m, o_ref):
    x = x_ref[...]
    ef = ef_ref[...]
    deg = (dg_ref[0] + dg_ref[1])[:, 0:1]
    norm = jnp.where(deg > 0.5, 1.0 / deg, 0.0)
    u = jnp.maximum((ef * x) @ w1m[...] + b1m[...], 0.0)
    out = u @ w2m[...] + b2m[...]
    bscal = x[:, D - 1:D] * ef
    o_ref[...] = norm * jnp.where(_col_is_last(out.shape), bscal, out)


def _c2v_msg(x, ef, dg, wm):
    return pl.pallas_call(
        _c2v_msg_body,
        grid=(GB,),
        in_specs=[_rows(D), _rows(1), _DEG_SPEC] + [_full(w.shape) for w in wm],
        out_specs=_rows(D),
        out_shape=jax.ShapeDtypeStruct((N, D), jnp.float32),
    )(x, ef, dg, *wm)


def _c2v_upd_body(ag_ref, xd_ref, w1h, b1h, w2h, b2h, wr, br, o_ref):
    aggr = ag_ref[0] + ag_ref[1]
    xd = xd_ref[...]
    t = jax.nn.sigmoid(xd @ w1h[...] + b1h[...])
    a = (t @ w2h[...] + b2h[...])[:, 0:1]
    main = jnp.maximum(aggr + xd @ wr[...] + br[...], 0.0)
    last = a * aggr[:, D - 1:D]
    o_ref[...] = jnp.where(_col_is_last(main.shape), last, main)


def _c2v_upd(ag, xd, wh, wr, br):
    return pl.pallas_call(
        _c2v_upd_body,
        grid=(GB,),
        in_specs=[_AGG_SPEC, _rows(D)] + [_full(w.shape) for w in wh] +
                 [_full(wr.shape), _full(br.shape)],
        out_specs=_rows(D),
        out_shape=jax.ShapeDtypeStruct((N, D), jnp.float32),
    )(ag, xd, *wh, wr, br)


def _head_body(x_ref, *refs):
    o_ref = refs[-1]
    ws = refs[:-1]
    x = x_ref[...]
    for i in range(5):
        x = jnp.maximum(x @ ws[2 * i][...] + ws[2 * i + 1][...], 0.0)
    lg = x @ ws[10][...] + ws[11][...]
    o_ref[...] = jax.nn.log_softmax(lg, axis=1)


def _head(x, ws):
    return pl.pallas_call(
        _head_body,
        grid=(GB,),
        in_specs=[_rows(D)] + [_full(w.shape) for w in ws],
        out_specs=_rows(2),
        out_shape=jax.ShapeDtypeStruct((N, 2), jnp.float32),
    )(x, *ws)


def _pad_mlp2(p, pre):
    w1 = jnp.pad(p[pre + '_W1'], ((0, 0), (0, 1)))
    b1 = jnp.pad(p[pre + '_b1'], (0, 1)).reshape(1, D)
    w2 = jnp.pad(p[pre + '_W2'], ((0, 1), (0, 1)))
    b2 = jnp.pad(p[pre + '_b2'], (0, 1)).reshape(1, D)
    return (w1, b1, w2, b2)


def _pad_h2v(p, l):
    w1 = jnp.pad(p['h2v%d_W1' % l], ((0, 0), (0, 1)))
    b1 = jnp.pad(p['h2v%d_b1' % l], (0, 1)).reshape(1, D)
    w2 = jnp.pad(p['h2v%d_W2' % l], ((0, 1), (0, D - 1)))
    b2 = jnp.pad(p['h2v%d_b2' % l], (0, D - 1)).reshape(1, D)
    return (w1, b1, w2, b2)


def _pad_root(p, pre):
    wr = jnp.pad(p[pre + '_root'], ((0, 0), (0, 1)))
    br = jnp.pad(p[pre + '_bias'], (0, 1)).reshape(1, D)
    return wr, br


def _edge_layouts(src, dst):
    pad0 = jnp.zeros((PADN,), jnp.int32)
    pads = jnp.full((PADN,), SENT, jnp.int32)
    g_src = jnp.concatenate([src, pad0]).reshape(ROWS, KM)
    g_dst = jnp.concatenate([dst, pads]).reshape(ROWS, KM)
    d_src = jnp.concatenate([src, pads]).reshape(ROWS, KM)
    return g_src, g_dst, d_src


def _impl(vnf, cnf, eiv, efv, rhs, eic, efc, asums, params):
    del asums
    p = params
    sv = eiv[0].astype(jnp.int32)
    dv = eiv[1].astype(jnp.int32)
    sc = eic[0].astype(jnp.int32)
    dc = eic[1].astype(jnp.int32)
    svg, dvg, svd = _edge_layouts(sv, dv)
    scg, dcg, scd = _edge_layouts(sc, dc)
    zero_d = jnp.zeros((NPAD, D), jnp.float32)
    zero_w = jnp.zeros((NPAD, DW), jnp.float32)
    ones_w = jnp.ones((KM, DW), jnp.float32)
    efv = efv.astype(jnp.float32)
    efc = efc.astype(jnp.float32)
    rhs2 = rhs.astype(jnp.float32).reshape(N, 1)

    deg_v = _deg(svd, ones_w, zero_w)
    deg_c = _deg(scd, ones_w, zero_w)

    v0 = _mlp2_tc(vnf, *_pad_mlp2(p, 'con_mlp'))
    c0 = _mlp2_tc(cnf, *_pad_mlp2(p, 'var_mlp'))

    x_src, old_cons, vars_ = v0, c0, v0
    for l in (1, 2, 3, 4):
        wh = _pad_h2v(p, l)
        wm_v = _pad_mlp2(p, 'v2c%d' % l)
        mv = _v2c_msg(x_src, efv, deg_v, wh, wm_v)
        ag = _spmm(mv, svg, dvg, zero_d)
        cons = _v2c_upd(ag, old_cons, rhs2, *_pad_root(p, 'v2c%d' % l))

        wm_c = _pad_mlp2(p, 'c2v%d' % l)
        mc = _c2v_msg(cons, efc, deg_c, wm_c)
        ag2 = _spmm(mc, scg, dcg, zero_d)
        wr, br = _pad_root(p, 'c2v%d' % l)
        vars_ = _c2v_upd(ag2, vars_, wh, wr, br)
        x_src, old_cons = vars_, cons

    ws = []
    for i in range(1, 6):
        ws += [p['fc%d_W' % i], p['fc%d_b' % i].reshape(1, D)]
    ws += [p['fc6_W'], p['fc6_b'].reshape(1, 2)]
    return _head(vars_, ws)


def kernel(var_node_features, con_node_features, edge_index_var, edge_features_var, rhs, edge_index_con, edge_features_con, asums, params):
    return _impl(var_node_features, con_node_features, edge_index_var,
                 edge_features_var, rhs, edge_index_con, edge_features_con,
                 asums, params)


# speedup vs baseline: 4.4978x; 1.1646x over previous
import functools

import jax
import jax.numpy as jnp
from jax import lax
from jax.experimental import pallas as pl
from jax.experimental.pallas import tpu as pltpu
from jax.experimental.pallas import tpu_sc as plsc

N = 25000
NE = 800000
D = 32
DW = 8
KM = 128
NW = 32
ROWS = 6400
RW = ROWS // NW
PADN = ROWS * KM - NE
SENT = N
NPAD = N + 8
BB = 1000
GB = N // BB


@functools.lru_cache(maxsize=None)
def _build_spmm():
    mesh = plsc.VectorSubcoreMesh(core_axis_name="c", subcore_axis_name="s")
    return functools.partial(
        pl.kernel,
        out_type=jax.ShapeDtypeStruct((2, NPAD, D), jnp.float32),
        mesh=mesh,
        scratch_types=[
            pltpu.VMEM((RW, KM), jnp.int32),
            pltpu.VMEM((RW, KM), jnp.int32),
            pltpu.VMEM((KM, D), jnp.float32),
            pltpu.VMEM((KM, D), jnp.float32),
            pltpu.VMEM_SHARED((NPAD, D), jnp.float32),
            pltpu.SemaphoreType.DMA,
            pltpu.SemaphoreType.DMA,
            pltpu.SemaphoreType.DMA,
            pltpu.SemaphoreType.DMA,
        ],
        compiler_params=pltpu.CompilerParams(use_tc_tiling_on_sc=False),
    )(_spmm_body)


def _spmm(m, srcg, dstg, zero):
    return _build_spmm()(m, srcg, dstg, zero)


def _spmm_body(m_hbm, srcg_hbm, dstg_hbm, zero_hbm, out_hbm, idx_v, didx_v,
               rows0, rows1, acc, semg0, semg1, sems0, sems1):
    c = lax.axis_index("c")
    s = lax.axis_index("s")
    wid = s * 2 + c
    base = wid * RW
    pltpu.sync_copy(srcg_hbm.at[pl.ds(base, RW)], idx_v)
    pltpu.sync_copy(dstg_hbm.at[pl.ds(base, RW)], didx_v)

    @pl.when(s == 0)
    def _zero():
        pltpu.sync_copy(zero_hbm, acc)

    plsc.subcore_barrier()
    bufs = ((rows0, semg0, sems0), (rows1, semg1, sems1))

    def g_wait(rv, sg):
        pltpu.make_async_copy(m_hbm.at[pl.ds(0, KM)], rv, sg).wait()

    def s_wait(rv, ss):
        pltpu.make_async_copy(m_hbm.at[pl.ds(0, KM)], rv, ss).wait()

    pltpu.async_copy(m_hbm.at[idx_v.at[0]], rows0, semg0)
    pltpu.async_copy(m_hbm.at[idx_v.at[1]], rows1, semg1)

    def body(j, carry):
        for b, (rv, sg, ss) in enumerate(bufs):
            i = j * 2 + b
            g_wait(rv, sg)
            pltpu.async_copy(rv, acc.at[didx_v.at[i]], ss, add=True)
        for b, (rv, sg, ss) in enumerate(bufs):
            i = j * 2 + b
            s_wait(rv, ss)

            @pl.when(i + 2 < RW)
            def _prefetch():
                pltpu.async_copy(m_hbm.at[idx_v.at[i + 2]], rv, sg)

        return carry

    lax.fori_loop(0, RW // 2, body, 0)
    plsc.subcore_barrier()

    @pl.when(s == 0)
    def _writeback():
        pltpu.sync_copy(acc, out_hbm.at[c])


@functools.lru_cache(maxsize=None)
def _build_deg():
    mesh = plsc.VectorSubcoreMesh(core_axis_name="c", subcore_axis_name="s")
    return functools.partial(
        pl.kernel,
        out_type=jax.ShapeDtypeStruct((2, NPAD, DW), jnp.float32),
        mesh=mesh,
        scratch_types=[
            pltpu.VMEM((RW, KM), jnp.int32),
            pltpu.VMEM((KM, DW), jnp.float32),
            pltpu.VMEM_SHARED((NPAD, DW), jnp.float32),
        ],
        compiler_params=pltpu.CompilerParams(use_tc_tiling_on_sc=False),
    )(_deg_body)


def _deg(srcd, ones_w, zero_w):
    return _build_deg()(srcd, ones_w, zero_w)


def _deg_body(srcd_hbm, ones_hbm, zero_hbm, out_hbm, idx_v, ones_v, acc):
    c = lax.axis_index("c")
    s = lax.axis_index("s")
    wid = s * 2 + c
    base = wid * RW
    pltpu.sync_copy(srcd_hbm.at[pl.ds(base, RW)], idx_v)
    pltpu.sync_copy(ones_hbm, ones_v)

    @pl.when(s == 0)
    def _zero():
        pltpu.sync_copy(zero_hbm, acc)

    plsc.subcore_barrier()

    def body(i, carry):
        pltpu.sync_copy(ones_v, acc.at[idx_v.at[i]], add=True)
        return carry

    lax.fori_loop(0, RW, body, 0)
    plsc.subcore_barrier()

    @pl.when(s == 0)
    def _writeback():
        pltpu.sync_copy(acc, out_hbm.at[c])


def _full(shape):
    return pl.BlockSpec(shape, lambda i: tuple(0 for _ in shape))


def _rows(cols, b=BB):
    return pl.BlockSpec((b, cols), lambda i: (i, 0))


_AGG_SPEC = pl.BlockSpec((2, BB, D), lambda i: (0, i, 0))
_DEG_SPEC = pl.BlockSpec((2, BB, DW), lambda i: (0, i, 0))


def _col_is_last(shape):
    return lax.broadcasted_iota(jnp.int32, shape, 1) == (D - 1)


def _mlp2_body(x_ref, w1_ref, b1_ref, w2_ref, b2_ref, o_ref):
    h = jnp.maximum(x_ref[...] @ w1_ref[...] + b1_ref[...], 0.0)
    o_ref[...] = h @ w2_ref[...] + b2_ref[...]


def _mlp2_tc(x, w1, b1, w2, b2):
    cin = x.shape[1]
    return pl.pallas_call(
        _mlp2_body,
        grid=(GB,),
        in_specs=[_rows(cin), _full(w1.shape), _full(b1.shape),
                  _full(w2.shape), _full(b2.shape)],
        out_specs=_rows(D),
        out_shape=jax.ShapeDtypeStruct((N, D), jnp.float32),
    )(x, w1, b1, w2, b2)


def _v2c_msg_body(x_ref, ef_ref, dg_ref, w1h, b1h, w2h, b2h, w1m, b1m, w2m, b2m, o_ref):
    x = x_ref[...]
    ef = ef_ref[...]
    deg = (dg_ref[0] + dg_ref[1])[:, 0:1]
    norm = jnp.where(deg > 0.5, 1.0 / deg, 0.0)
    t = jax.nn.sigmoid(x @ w1h[...] + b1h[...])
    va = (t @ w2h[...] + b2h[...])[:, 0:1] * ef
    u = jnp.maximum((ef * x) @ w1m[...] + b1m[...], 0.0)
    out = (u @ w2m[...] + b2m[...]) * norm
    o_ref[...] = jnp.where(_col_is_last(out.shape), va, out)


def _v2c_msg(x, ef, dg, wh, wm):
    return pl.pallas_call(
        _v2c_msg_body,
        grid=(GB,),
        in_specs=[_rows(D), _rows(1), _DEG_SPEC] +
                 [_full(w.shape) for w in wh] + [_full(w.shape) for w in wm],
        out_specs=_rows(D),
        out_shape=jax.ShapeDtypeStruct((N, D), jnp.float32),
    )(x, ef, dg, *wh, *wm)


def _v2c_upd_body(ag_ref, old_ref, rhs_ref, wr, br, o_ref):
    aggr = ag_ref[0] + ag_ref[1]
    main = jnp.maximum(aggr + old_ref[...] @ wr[...] + br[...], 0.0)
    last = aggr[:, D - 1:D] - rhs_ref[...]
    o_ref[...] = jnp.where(_col_is_last(main.shape), last, main)


def _v2c_upd(ag, old, rhs2, wr, br):
    return pl.pallas_call(
        _v2c_upd_body,
        grid=(GB,),
        in_specs=[_AGG_SPEC, _rows(D), _rows(1), _full(wr.shape), _full(br.shape)],
        out_specs=_rows(D),
        out_shape=jax.ShapeDtypeStruct((N, D), jnp.float32),
    )(ag, old, rhs2, wr, br)


def _c2v_msg_body(x_ref, ef_ref, dg_ref, w1m, b1m, w2m, b2m, o_ref):
    x = x_ref[...]
    ef = ef_ref[...]
    deg = (dg_ref[0] + dg_ref[1])[:, 0:1]
    norm = jnp.where(deg > 0.5, 1.0 / deg, 0.0)
    u = jnp.maximum((ef * x) @ w1m[...] + b1m[...], 0.0)
    out = u @ w2m[...] + b2m[...]
    bscal = x[:, D - 1:D] * ef
    o_ref[...] = norm * jnp.where(_col_is_last(out.shape), bscal, out)


def _c2v_msg(x, ef, dg, wm):
    return pl.pallas_call(
        _c2v_msg_body,
        grid=(GB,),
        in_specs=[_rows(D), _rows(1), _DEG_SPEC] + [_full(w.shape) for w in wm],
        out_specs=_rows(D),
        out_shape=jax.ShapeDtypeStruct((N, D), jnp.float32),
    )(x, ef, dg, *wm)


def _c2v_upd_body(ag_ref, xd_ref, w1h, b1h, w2h, b2h, wr, br, o_ref):
    aggr = ag_ref[0] + ag_ref[1]
    xd = xd_ref[...]
    t = jax.nn.sigmoid(xd @ w1h[...] + b1h[...])
    a = (t @ w2h[...] + b2h[...])[:, 0:1]
    main = jnp.maximum(aggr + xd @ wr[...] + br[...], 0.0)
    last = a * aggr[:, D - 1:D]
    o_ref[...] = jnp.where(_col_is_last(main.shape), last, main)


def _c2v_upd(ag, xd, wh, wr, br):
    return pl.pallas_call(
        _c2v_upd_body,
        grid=(GB,),
        in_specs=[_AGG_SPEC, _rows(D)] + [_full(w.shape) for w in wh] +
                 [_full(wr.shape), _full(br.shape)],
        out_specs=_rows(D),
        out_shape=jax.ShapeDtypeStruct((N, D), jnp.float32),
    )(ag, xd, *wh, wr, br)


def _head_body(x_ref, *refs):
    o_ref = refs[-1]
    ws = refs[:-1]
    x = x_ref[...]
    for i in range(5):
        x = jnp.maximum(x @ ws[2 * i][...] + ws[2 * i + 1][...], 0.0)
    lg = x @ ws[10][...] + ws[11][...]
    o_ref[...] = jax.nn.log_softmax(lg, axis=1)


def _head(x, ws):
    return pl.pallas_call(
        _head_body,
        grid=(GB,),
        in_specs=[_rows(D)] + [_full(w.shape) for w in ws],
        out_specs=_rows(2),
        out_shape=jax.ShapeDtypeStruct((N, 2), jnp.float32),
    )(x, *ws)


def _pad_mlp2(p, pre):
    w1 = jnp.pad(p[pre + '_W1'], ((0, 0), (0, 1)))
    b1 = jnp.pad(p[pre + '_b1'], (0, 1)).reshape(1, D)
    w2 = jnp.pad(p[pre + '_W2'], ((0, 1), (0, 1)))
    b2 = jnp.pad(p[pre + '_b2'], (0, 1)).reshape(1, D)
    return (w1, b1, w2, b2)


def _pad_h2v(p, l):
    w1 = jnp.pad(p['h2v%d_W1' % l], ((0, 0), (0, 1)))
    b1 = jnp.pad(p['h2v%d_b1' % l], (0, 1)).reshape(1, D)
    w2 = jnp.pad(p['h2v%d_W2' % l], ((0, 1), (0, D - 1)))
    b2 = jnp.pad(p['h2v%d_b2' % l], (0, D - 1)).reshape(1, D)
    return (w1, b1, w2, b2)


def _pad_root(p, pre):
    wr = jnp.pad(p[pre + '_root'], ((0, 0), (0, 1)))
    br = jnp.pad(p[pre + '_bias'], (0, 1)).reshape(1, D)
    return wr, br


def _edge_layouts(src, dst):
    pad0 = jnp.zeros((PADN,), jnp.int32)
    pads = jnp.full((PADN,), SENT, jnp.int32)
    g_src = jnp.concatenate([src, pad0]).reshape(ROWS, KM)
    g_dst = jnp.concatenate([dst, pads]).reshape(ROWS, KM)
    d_src = jnp.concatenate([src, pads]).reshape(ROWS, KM)
    return g_src, g_dst, d_src


def _impl(vnf, cnf, eiv, efv, rhs, eic, efc, asums, params):
    del asums
    p = params
    sv = eiv[0].astype(jnp.int32)
    dv = eiv[1].astype(jnp.int32)
    sc = eic[0].astype(jnp.int32)
    dc = eic[1].astype(jnp.int32)
    svg, dvg, svd = _edge_layouts(sv, dv)
    scg, dcg, scd = _edge_layouts(sc, dc)
    zero_d = jnp.zeros((NPAD, D), jnp.float32)
    zero_w = jnp.zeros((NPAD, DW), jnp.float32)
    ones_w = jnp.ones((KM, DW), jnp.float32)
    efv = efv.astype(jnp.float32)
    efc = efc.astype(jnp.float32)
    rhs2 = rhs.astype(jnp.float32).reshape(N, 1)

    deg_v = _deg(svd, ones_w, zero_w)
    deg_c = _deg(scd, ones_w, zero_w)

    v0 = _mlp2_tc(vnf, *_pad_mlp2(p, 'con_mlp'))
    c0 = _mlp2_tc(cnf, *_pad_mlp2(p, 'var_mlp'))

    x_src, old_cons, vars_ = v0, c0, v0
    for l in (1, 2, 3, 4):
        wh = _pad_h2v(p, l)
        wm_v = _pad_mlp2(p, 'v2c%d' % l)
        mv = _v2c_msg(x_src, efv, deg_v, wh, wm_v)
        ag = _spmm(mv, svg, dvg, zero_d)
        cons = _v2c_upd(ag, old_cons, rhs2, *_pad_root(p, 'v2c%d' % l))

        wm_c = _pad_mlp2(p, 'c2v%d' % l)
        mc = _c2v_msg(cons, efc, deg_c, wm_c)
        ag2 = _spmm(mc, scg, dcg, zero_d)
        wr, br = _pad_root(p, 'c2v%d' % l)
        vars_ = _c2v_upd(ag2, vars_, wh, wr, br)
        x_src, old_cons = vars_, cons

    ws = []
    for i in range(1, 6):
        ws += [p['fc%d_W' % i], p['fc%d_b' % i].reshape(1, D)]
    ws += [p['fc6_W'], p['fc6_b'].reshape(1, 2)]
    return _head(vars_, ws)


def kernel(var_node_features, con_node_features, edge_index_var, edge_features_var, rhs, edge_index_con, edge_features_con, asums, params):
    return _impl(var_node_features, con_node_features, edge_index_var,
                 edge_features_var, rhs, edge_index_con, edge_features_con,
                 asums, params)


# speedup vs baseline: 5.2375x; 1.1645x over previous
import functools

import jax
import jax.numpy as jnp
from jax import lax
from jax.experimental import pallas as pl
from jax.experimental.pallas import tpu as pltpu
from jax.experimental.pallas import tpu_sc as plsc

N = 25000
NE = 800000
D = 32
DW = 8
KM = 128
NW = 32
ROWS = 6400
RW = ROWS // NW
PADN = ROWS * KM - NE
SENT = N
NPAD = N + 8
BB = 1000
GB = N // BB


@functools.lru_cache(maxsize=None)
def _build_spmm():
    mesh = plsc.VectorSubcoreMesh(core_axis_name="c", subcore_axis_name="s")
    return functools.partial(
        pl.kernel,
        out_type=jax.ShapeDtypeStruct((2, NPAD, D), jnp.float32),
        mesh=mesh,
        scratch_types=[
            pltpu.VMEM((RW, KM), jnp.int32),
            pltpu.VMEM((RW, KM), jnp.int32),
            pltpu.VMEM((KM, D), jnp.float32),
            pltpu.VMEM((KM, D), jnp.float32),
            pltpu.VMEM_SHARED((NPAD, D), jnp.float32),
            pltpu.SemaphoreType.DMA,
            pltpu.SemaphoreType.DMA,
            pltpu.SemaphoreType.DMA,
            pltpu.SemaphoreType.DMA,
        ],
        compiler_params=pltpu.CompilerParams(use_tc_tiling_on_sc=False),
    )(_spmm_body)


def _spmm(m, srcg, dstg, zero):
    return _build_spmm()(m, srcg, dstg, zero)


def _spmm_body(m_hbm, srcg_hbm, dstg_hbm, zero_hbm, out_hbm, idx_v, didx_v,
               rows0, rows1, acc, semg0, semg1, sems0, sems1):
    c = lax.axis_index("c")
    s = lax.axis_index("s")
    wid = s * 2 + c
    base = wid * RW
    pltpu.sync_copy(srcg_hbm.at[pl.ds(base, RW)], idx_v)
    pltpu.sync_copy(dstg_hbm.at[pl.ds(base, RW)], didx_v)

    @pl.when(s == 0)
    def _zero():
        pltpu.sync_copy(zero_hbm, acc)

    plsc.subcore_barrier()
    bufs = ((rows0, semg0, sems0), (rows1, semg1, sems1))

    def g_wait(rv, sg):
        pltpu.make_async_copy(m_hbm.at[pl.ds(0, KM)], rv, sg).wait()

    def s_wait(rv, ss):
        pltpu.make_async_copy(m_hbm.at[pl.ds(0, KM)], rv, ss).wait()

    pltpu.async_copy(m_hbm.at[idx_v.at[0]], rows0, semg0)
    pltpu.async_copy(m_hbm.at[idx_v.at[1]], rows1, semg1)

    def body(j, carry):
        for b, (rv, sg, ss) in enumerate(bufs):
            i = j * 2 + b
            g_wait(rv, sg)
            pltpu.async_copy(rv, acc.at[didx_v.at[i]], ss, add=True)
        for b, (rv, sg, ss) in enumerate(bufs):
            i = j * 2 + b
            s_wait(rv, ss)

            @pl.when(i + 2 < RW)
            def _prefetch():
                pltpu.async_copy(m_hbm.at[idx_v.at[i + 2]], rv, sg)

        return carry

    lax.fori_loop(0, RW // 2, body, 0)
    plsc.subcore_barrier()

    @pl.when(s == 0)
    def _writeback():
        pltpu.sync_copy(acc, out_hbm.at[c])


@functools.lru_cache(maxsize=None)
def _build_deg():
    mesh = plsc.VectorSubcoreMesh(core_axis_name="c", subcore_axis_name="s")
    return functools.partial(
        pl.kernel,
        out_type=jax.ShapeDtypeStruct((2, NPAD, DW), jnp.float32),
        mesh=mesh,
        scratch_types=[
            pltpu.VMEM((RW, KM), jnp.int32),
            pltpu.VMEM((KM, DW), jnp.float32),
            pltpu.VMEM_SHARED((NPAD, DW), jnp.float32),
            pltpu.SemaphoreType.DMA,
        ],
        compiler_params=pltpu.CompilerParams(use_tc_tiling_on_sc=False),
    )(_deg_body)


def _deg(srcd, ones_w, zero_w):
    return _build_deg()(srcd, ones_w, zero_w)


def _deg_body(srcd_hbm, ones_hbm, zero_hbm, out_hbm, idx_v, ones_v, acc, sem):
    c = lax.axis_index("c")
    s = lax.axis_index("s")
    wid = s * 2 + c
    base = wid * RW
    pltpu.sync_copy(srcd_hbm.at[pl.ds(base, RW)], idx_v)
    pltpu.sync_copy(ones_hbm, ones_v)

    @pl.when(s == 0)
    def _zero():
        pltpu.sync_copy(zero_hbm, acc)

    plsc.subcore_barrier()

    def body(i, carry):
        @pl.when(i >= 4)
        def _():
            pltpu.make_async_copy(ones_hbm, ones_v, sem).wait()

        pltpu.async_copy(ones_v, acc.at[idx_v.at[i]], sem, add=True)
        return carry

    lax.fori_loop(0, RW, body, 0)
    for _ in range(4):
        pltpu.make_async_copy(ones_hbm, ones_v, sem).wait()
    plsc.subcore_barrier()

    @pl.when(s == 0)
    def _writeback():
        pltpu.sync_copy(acc, out_hbm.at[c])


def _full(shape):
    return pl.BlockSpec(shape, lambda i: tuple(0 for _ in shape))


def _rows(cols, b=BB):
    return pl.BlockSpec((b, cols), lambda i: (i, 0))


_AGG_SPEC = pl.BlockSpec((2, BB, D), lambda i: (0, i, 0))
_DEG_SPEC = pl.BlockSpec((2, BB, DW), lambda i: (0, i, 0))


def _col_is_last(shape):
    return lax.broadcasted_iota(jnp.int32, shape, 1) == (D - 1)


def _mlp2_body(x_ref, w1_ref, b1_ref, w2_ref, b2_ref, o_ref):
    h = jnp.maximum(x_ref[...] @ w1_ref[...] + b1_ref[...], 0.0)
    o_ref[...] = h @ w2_ref[...] + b2_ref[...]


def _mlp2_tc(x, w1, b1, w2, b2):
    cin = x.shape[1]
    return pl.pallas_call(
        _mlp2_body,
        grid=(GB,),
        in_specs=[_rows(cin), _full(w1.shape), _full(b1.shape),
                  _full(w2.shape), _full(b2.shape)],
        out_specs=_rows(D),
        out_shape=jax.ShapeDtypeStruct((N, D), jnp.float32),
    )(x, w1, b1, w2, b2)


def _v2c_msg_body(x_ref, ef_ref, dg_ref, w1h, b1h, w2h, b2h, w1m, b1m, w2m, b2m, o_ref):
    x = x_ref[...]
    ef = ef_ref[...]
    deg = (dg_ref[0] + dg_ref[1])[:, 0:1]
    norm = jnp.where(deg > 0.5, 1.0 / deg, 0.0)
    t = jax.nn.sigmoid(x @ w1h[...] + b1h[...])
    va = (t @ w2h[...] + b2h[...])[:, 0:1] * ef
    u = jnp.maximum((ef * x) @ w1m[...] + b1m[...], 0.0)
    out = (u @ w2m[...] + b2m[...]) * norm
    o_ref[...] = jnp.where(_col_is_last(out.shape), va, out)


def _v2c_msg(x, ef, dg, wh, wm):
    return pl.pallas_call(
        _v2c_msg_body,
        grid=(GB,),
        in_specs=[_rows(D), _rows(1), _DEG_SPEC] +
                 [_full(w.shape) for w in wh] + [_full(w.shape) for w in wm],
        out_specs=_rows(D),
        out_shape=jax.ShapeDtypeStruct((N, D), jnp.float32),
    )(x, ef, dg, *wh, *wm)


def _v2c_upd_body(ag_ref, old_ref, rhs_ref, wr, br, o_ref):
    aggr = ag_ref[0] + ag_ref[1]
    main = jnp.maximum(aggr + old_ref[...] @ wr[...] + br[...], 0.0)
    last = aggr[:, D - 1:D] - rhs_ref[...]
    o_ref[...] = jnp.where(_col_is_last(main.shape), last, main)


def _v2c_upd(ag, old, rhs2, wr, br):
    return pl.pallas_call(
        _v2c_upd_body,
        grid=(GB,),
        in_specs=[_AGG_SPEC, _rows(D), _rows(1), _full(wr.shape), _full(br.shape)],
        out_specs=_rows(D),
        out_shape=jax.ShapeDtypeStruct((N, D), jnp.float32),
    )(ag, old, rhs2, wr, br)


def _c2v_msg_body(x_ref, ef_ref, dg_ref, w1m, b1m, w2m, b2m, o_ref):
    x = x_ref[...]
    ef = ef_ref[...]
    deg = (dg_ref[0] + dg_ref[1])[:, 0:1]
    norm = jnp.where(deg > 0.5, 1.0 / deg, 0.0)
    u = jnp.maximum((ef * x) @ w1m[...] + b1m[...], 0.0)
    out = u @ w2m[...] + b2m[...]
    bscal = x[:, D - 1:D] * ef
    o_ref[...] = norm * jnp.where(_col_is_last(out.shape), bscal, out)


def _c2v_msg(x, ef, dg, wm):
    return pl.pallas_call(
        _c2v_msg_body,
        grid=(GB,),
        in_specs=[_rows(D), _rows(1), _DEG_SPEC] + [_full(w.shape) for w in wm],
        out_specs=_rows(D),
        out_shape=jax.ShapeDtypeStruct((N, D), jnp.float32),
    )(x, ef, dg, *wm)


def _c2v_upd_body(ag_ref, xd_ref, w1h, b1h, w2h, b2h, wr, br, o_ref):
    aggr = ag_ref[0] + ag_ref[1]
    xd = xd_ref[...]
    t = jax.nn.sigmoid(xd @ w1h[...] + b1h[...])
    a = (t @ w2h[...] + b2h[...])[:, 0:1]
    main = jnp.maximum(aggr + xd @ wr[...] + br[...], 0.0)
    last = a * aggr[:, D - 1:D]
    o_ref[...] = jnp.where(_col_is_last(main.shape), last, main)


def _c2v_upd(ag, xd, wh, wr, br):
    return pl.pallas_call(
        _c2v_upd_body,
        grid=(GB,),
        in_specs=[_AGG_SPEC, _rows(D)] + [_full(w.shape) for w in wh] +
                 [_full(wr.shape), _full(br.shape)],
        out_specs=_rows(D),
        out_shape=jax.ShapeDtypeStruct((N, D), jnp.float32),
    )(ag, xd, *wh, wr, br)


def _embed2_body(v_ref, c_ref, vw1, vb1, vw2, vb2, cw1, cb1, cw2, cb2, ov, oc):
    hv = jnp.maximum(v_ref[...] @ vw1[...] + vb1[...], 0.0)
    ov[...] = hv @ vw2[...] + vb2[...]
    hc = jnp.maximum(c_ref[...] @ cw1[...] + cb1[...], 0.0)
    oc[...] = hc @ cw2[...] + cb2[...]


def _embed2(vnf, cnf, wv, wc):
    return pl.pallas_call(
        _embed2_body,
        grid=(GB,),
        in_specs=[_rows(2), _rows(2)] + [_full(w.shape) for w in wv] +
                 [_full(w.shape) for w in wc],
        out_specs=(_rows(D), _rows(D)),
        out_shape=(jax.ShapeDtypeStruct((N, D), jnp.float32),
                   jax.ShapeDtypeStruct((N, D), jnp.float32)),
    )(vnf, cnf, *wv, *wc)


def _upd_cmsg_body(ag_ref, old_ref, rhs_ref, wr, br, ef_ref, dg_ref,
                   w1m, b1m, w2m, b2m, ocons, omc):
    aggr = ag_ref[0] + ag_ref[1]
    main = jnp.maximum(aggr + old_ref[...] @ wr[...] + br[...], 0.0)
    last = aggr[:, D - 1:D] - rhs_ref[...]
    cons = jnp.where(_col_is_last(main.shape), last, main)
    ocons[...] = cons
    ef = ef_ref[...]
    deg = (dg_ref[0] + dg_ref[1])[:, 0:1]
    norm = jnp.where(deg > 0.5, 1.0 / deg, 0.0)
    u = jnp.maximum((ef * cons) @ w1m[...] + b1m[...], 0.0)
    out = u @ w2m[...] + b2m[...]
    bscal = cons[:, D - 1:D] * ef
    omc[...] = norm * jnp.where(_col_is_last(out.shape), bscal, out)


def _upd_cmsg(ag, old, rhs2, wr, br, ef, dg, wm):
    return pl.pallas_call(
        _upd_cmsg_body,
        grid=(GB,),
        in_specs=[_AGG_SPEC, _rows(D), _rows(1), _full(wr.shape),
                  _full(br.shape), _rows(1), _DEG_SPEC] +
                 [_full(w.shape) for w in wm],
        out_specs=(_rows(D), _rows(D)),
        out_shape=(jax.ShapeDtypeStruct((N, D), jnp.float32),
                   jax.ShapeDtypeStruct((N, D), jnp.float32)),
    )(ag, old, rhs2, wr, br, ef, dg, *wm)


def _cupd_vmsg_body(ag_ref, xd_ref, w1h, b1h, w2h, b2h, wr, br, ef_ref, dg_ref,
                    nw1h, nb1h, nw2h, nb2h, nw1m, nb1m, nw2m, nb2m, ovars, omv):
    aggr = ag_ref[0] + ag_ref[1]
    xd = xd_ref[...]
    t = jax.nn.sigmoid(xd @ w1h[...] + b1h[...])
    a = (t @ w2h[...] + b2h[...])[:, 0:1]
    main = jnp.maximum(aggr + xd @ wr[...] + br[...], 0.0)
    last = a * aggr[:, D - 1:D]
    vrs = jnp.where(_col_is_last(main.shape), last, main)
    ovars[...] = vrs
    ef = ef_ref[...]
    deg = (dg_ref[0] + dg_ref[1])[:, 0:1]
    norm = jnp.where(deg > 0.5, 1.0 / deg, 0.0)
    t2 = jax.nn.sigmoid(vrs @ nw1h[...] + nb1h[...])
    va = (t2 @ nw2h[...] + nb2h[...])[:, 0:1] * ef
    u = jnp.maximum((ef * vrs) @ nw1m[...] + nb1m[...], 0.0)
    out = (u @ nw2m[...] + nb2m[...]) * norm
    omv[...] = jnp.where(_col_is_last(out.shape), va, out)


def _cupd_vmsg(ag, xd, wh, wr, br, ef, dg, nwh, nwm):
    return pl.pallas_call(
        _cupd_vmsg_body,
        grid=(GB,),
        in_specs=[_AGG_SPEC, _rows(D)] + [_full(w.shape) for w in wh] +
                 [_full(wr.shape), _full(br.shape), _rows(1), _DEG_SPEC] +
                 [_full(w.shape) for w in nwh] + [_full(w.shape) for w in nwm],
        out_specs=(_rows(D), _rows(D)),
        out_shape=(jax.ShapeDtypeStruct((N, D), jnp.float32),
                   jax.ShapeDtypeStruct((N, D), jnp.float32)),
    )(ag, xd, *wh, wr, br, ef, dg, *nwh, *nwm)


def _cupd_head_body(ag_ref, xd_ref, w1h, b1h, w2h, b2h, wr, br, *refs):
    o_ref = refs[-1]
    ws = refs[:-1]
    aggr = ag_ref[0] + ag_ref[1]
    xd = xd_ref[...]
    t = jax.nn.sigmoid(xd @ w1h[...] + b1h[...])
    a = (t @ w2h[...] + b2h[...])[:, 0:1]
    main = jnp.maximum(aggr + xd @ wr[...] + br[...], 0.0)
    last = a * aggr[:, D - 1:D]
    x = jnp.where(_col_is_last(main.shape), last, main)
    for i in range(5):
        x = jnp.maximum(x @ ws[2 * i][...] + ws[2 * i + 1][...], 0.0)
    lg = x @ ws[10][...] + ws[11][...]
    o_ref[...] = jax.nn.log_softmax(lg, axis=1)


def _cupd_head(ag, xd, wh, wr, br, ws):
    return pl.pallas_call(
        _cupd_head_body,
        grid=(GB,),
        in_specs=[_AGG_SPEC, _rows(D)] + [_full(w.shape) for w in wh] +
                 [_full(wr.shape), _full(br.shape)] +
                 [_full(w.shape) for w in ws],
        out_specs=_rows(2),
        out_shape=jax.ShapeDtypeStruct((N, 2), jnp.float32),
    )(ag, xd, *wh, wr, br, *ws)


def _head_body(x_ref, *refs):
    o_ref = refs[-1]
    ws = refs[:-1]
    x = x_ref[...]
    for i in range(5):
        x = jnp.maximum(x @ ws[2 * i][...] + ws[2 * i + 1][...], 0.0)
    lg = x @ ws[10][...] + ws[11][...]
    o_ref[...] = jax.nn.log_softmax(lg, axis=1)


def _head(x, ws):
    return pl.pallas_call(
        _head_body,
        grid=(GB,),
        in_specs=[_rows(D)] + [_full(w.shape) for w in ws],
        out_specs=_rows(2),
        out_shape=jax.ShapeDtypeStruct((N, 2), jnp.float32),
    )(x, *ws)


def _pad_mlp2(p, pre):
    w1 = jnp.pad(p[pre + '_W1'], ((0, 0), (0, 1)))
    b1 = jnp.pad(p[pre + '_b1'], (0, 1)).reshape(1, D)
    w2 = jnp.pad(p[pre + '_W2'], ((0, 1), (0, 1)))
    b2 = jnp.pad(p[pre + '_b2'], (0, 1)).reshape(1, D)
    return (w1, b1, w2, b2)


def _pad_h2v(p, l):
    w1 = jnp.pad(p['h2v%d_W1' % l], ((0, 0), (0, 1)))
    b1 = jnp.pad(p['h2v%d_b1' % l], (0, 1)).reshape(1, D)
    w2 = jnp.pad(p['h2v%d_W2' % l], ((0, 1), (0, D - 1)))
    b2 = jnp.pad(p['h2v%d_b2' % l], (0, D - 1)).reshape(1, D)
    return (w1, b1, w2, b2)


def _pad_root(p, pre):
    wr = jnp.pad(p[pre + '_root'], ((0, 0), (0, 1)))
    br = jnp.pad(p[pre + '_bias'], (0, 1)).reshape(1, D)
    return wr, br


def _edge_layouts(src, dst):
    pad0 = jnp.zeros((PADN,), jnp.int32)
    pads = jnp.full((PADN,), SENT, jnp.int32)
    g_src = jnp.concatenate([src, pad0]).reshape(ROWS, KM)
    g_dst = jnp.concatenate([dst, pads]).reshape(ROWS, KM)
    d_src = jnp.concatenate([src, pads]).reshape(ROWS, KM)
    return g_src, g_dst, d_src


def _impl(vnf, cnf, eiv, efv, rhs, eic, efc, asums, params):
    del asums
    p = params
    sv = eiv[0].astype(jnp.int32)
    dv = eiv[1].astype(jnp.int32)
    sc = eic[0].astype(jnp.int32)
    dc = eic[1].astype(jnp.int32)
    svg, dvg, svd = _edge_layouts(sv, dv)
    scg, dcg, scd = _edge_layouts(sc, dc)
    zero_d = jnp.zeros((NPAD, D), jnp.float32)
    zero_w = jnp.zeros((NPAD, DW), jnp.float32)
    ones_w = jnp.ones((KM, DW), jnp.float32)
    efv = efv.astype(jnp.float32)
    efc = efc.astype(jnp.float32)
    rhs2 = rhs.astype(jnp.float32).reshape(N, 1)

    deg_v = _deg(svd, ones_w, zero_w)
    deg_c = _deg(scd, ones_w, zero_w)

    v0, c0 = _embed2(vnf, cnf, _pad_mlp2(p, 'con_mlp'), _pad_mlp2(p, 'var_mlp'))

    ws = []
    for i in range(1, 6):
        ws += [p['fc%d_W' % i], p['fc%d_b' % i].reshape(1, D)]
    ws += [p['fc6_W'], p['fc6_b'].reshape(1, 2)]

    mv = _v2c_msg(v0, efv, deg_v, _pad_h2v(p, 1), _pad_mlp2(p, 'v2c1'))
    old_cons, vars_ = c0, v0
    for l in (1, 2, 3, 4):
        wh = _pad_h2v(p, l)
        ag = _spmm(mv, svg, dvg, zero_d)
        cons, mc = _upd_cmsg(ag, old_cons, rhs2, *_pad_root(p, 'v2c%d' % l),
                             efc, deg_c, _pad_mlp2(p, 'c2v%d' % l))
        ag2 = _spmm(mc, scg, dcg, zero_d)
        wr, br = _pad_root(p, 'c2v%d' % l)
        if l < 4:
            vars_, mv = _cupd_vmsg(ag2, vars_, wh, wr, br, efv, deg_v,
                                   _pad_h2v(p, l + 1),
                                   _pad_mlp2(p, 'v2c%d' % (l + 1)))
            old_cons = cons
        else:
            return _cupd_head(ag2, vars_, wh, wr, br, ws)


def kernel(var_node_features, con_node_features, edge_index_var, edge_features_var, rhs, edge_index_con, edge_features_con, asums, params):
    return _impl(var_node_features, con_node_features, edge_index_var,
                 edge_features_var, rhs, edge_index_con, edge_features_con,
                 asums, params)


# speedup vs baseline: 5.5358x; 1.0569x over previous
import functools

import jax
import jax.numpy as jnp
from jax import lax
from jax.experimental import pallas as pl
from jax.experimental.pallas import tpu as pltpu
from jax.experimental.pallas import tpu_sc as plsc

N = 25000
NE = 800000
D = 32
DW = 8
KM = 128
NW = 32
ROWS = 6400
RW = ROWS // NW
PADN = ROWS * KM - NE
SENT = N
NPAD = N + 8
BB = 1000
GB = N // BB


@functools.lru_cache(maxsize=None)
def _build_spmm():
    mesh = plsc.VectorSubcoreMesh(core_axis_name="c", subcore_axis_name="s")
    return functools.partial(
        pl.kernel,
        out_type=jax.ShapeDtypeStruct((2, NPAD, D), jnp.float32),
        mesh=mesh,
        scratch_types=[
            pltpu.VMEM((RW, KM), jnp.int32),
            pltpu.VMEM((RW, KM), jnp.int32)] +
        [pltpu.VMEM((KM, D), jnp.float32) for _ in range(4)] +
        [pltpu.VMEM_SHARED((NPAD, D), jnp.float32)] +
        [pltpu.SemaphoreType.DMA for _ in range(8)],
        compiler_params=pltpu.CompilerParams(use_tc_tiling_on_sc=False),
    )(_spmm_body)


def _spmm(m, srcg, dstg, zero):
    return _build_spmm()(m, srcg, dstg, zero)


def _spmm_body(m_hbm, srcg_hbm, dstg_hbm, zero_hbm, out_hbm, idx_v, didx_v,
               *scr):
    rows = scr[0:4]
    acc = scr[4]
    semg = scr[5:9]
    sems = scr[9:13]
    c = lax.axis_index("c")
    s = lax.axis_index("s")
    wid = s * 2 + c
    base = wid * RW
    pltpu.sync_copy(srcg_hbm.at[pl.ds(base, RW)], idx_v)
    pltpu.sync_copy(dstg_hbm.at[pl.ds(base, RW)], didx_v)

    @pl.when(s == 0)
    def _zero():
        pltpu.sync_copy(zero_hbm, acc)

    plsc.subcore_barrier()
    nb = 4

    def dma_wait(rv, sm):
        pltpu.make_async_copy(m_hbm.at[pl.ds(0, KM)], rv, sm).wait()

    for b in range(nb):
        pltpu.async_copy(m_hbm.at[idx_v.at[b]], rows[b], semg[b])

    def body(j, carry):
        for b in range(nb):
            i = j * nb + b
            dma_wait(rows[b], semg[b])
            pltpu.async_copy(rows[b], acc.at[didx_v.at[i]], sems[b], add=True)
        for b in range(nb):
            i = j * nb + b
            dma_wait(rows[b], sems[b])

            @pl.when(i + nb < RW)
            def _prefetch():
                pltpu.async_copy(m_hbm.at[idx_v.at[i + nb]], rows[b], semg[b])

        return carry

    lax.fori_loop(0, RW // nb, body, 0)
    plsc.subcore_barrier()

    @pl.when(s == 0)
    def _writeback():
        pltpu.sync_copy(acc, out_hbm.at[c])


@functools.lru_cache(maxsize=None)
def _build_deg():
    mesh = plsc.VectorSubcoreMesh(core_axis_name="c", subcore_axis_name="s")
    return functools.partial(
        pl.kernel,
        out_type=jax.ShapeDtypeStruct((2, NPAD, DW), jnp.float32),
        mesh=mesh,
        scratch_types=[
            pltpu.VMEM((RW, KM), jnp.int32),
            pltpu.VMEM((KM, DW), jnp.float32),
            pltpu.VMEM_SHARED((NPAD, DW), jnp.float32),
            pltpu.SemaphoreType.DMA,
        ],
        compiler_params=pltpu.CompilerParams(use_tc_tiling_on_sc=False),
    )(_deg_body)


def _deg(srcd, ones_w, zero_w):
    return _build_deg()(srcd, ones_w, zero_w)


def _deg_body(srcd_hbm, ones_hbm, zero_hbm, out_hbm, idx_v, ones_v, acc, sem):
    c = lax.axis_index("c")
    s = lax.axis_index("s")
    wid = s * 2 + c
    base = wid * RW
    pltpu.sync_copy(srcd_hbm.at[pl.ds(base, RW)], idx_v)
    pltpu.sync_copy(ones_hbm, ones_v)

    @pl.when(s == 0)
    def _zero():
        pltpu.sync_copy(zero_hbm, acc)

    plsc.subcore_barrier()

    def body(i, carry):
        @pl.when(i >= 8)
        def _():
            pltpu.make_async_copy(ones_hbm, ones_v, sem).wait()

        pltpu.async_copy(ones_v, acc.at[idx_v.at[i]], sem, add=True)
        return carry

    lax.fori_loop(0, RW, body, 0)
    for _ in range(8):
        pltpu.make_async_copy(ones_hbm, ones_v, sem).wait()
    plsc.subcore_barrier()

    @pl.when(s == 0)
    def _writeback():
        pltpu.sync_copy(acc, out_hbm.at[c])


def _full(shape):
    return pl.BlockSpec(shape, lambda i: tuple(0 for _ in shape))


def _rows(cols, b=BB):
    return pl.BlockSpec((b, cols), lambda i: (i, 0))


_AGG_SPEC = pl.BlockSpec((2, BB, D), lambda i: (0, i, 0))
_DEG_SPEC = pl.BlockSpec((2, BB, DW), lambda i: (0, i, 0))


def _col_is_last(shape):
    return lax.broadcasted_iota(jnp.int32, shape, 1) == (D - 1)


def _mlp2_body(x_ref, w1_ref, b1_ref, w2_ref, b2_ref, o_ref):
    h = jnp.maximum(x_ref[...] @ w1_ref[...] + b1_ref[...], 0.0)
    o_ref[...] = h @ w2_ref[...] + b2_ref[...]


def _mlp2_tc(x, w1, b1, w2, b2):
    cin = x.shape[1]
    return pl.pallas_call(
        _mlp2_body,
        grid=(GB,),
        in_specs=[_rows(cin), _full(w1.shape), _full(b1.shape),
                  _full(w2.shape), _full(b2.shape)],
        out_specs=_rows(D),
        out_shape=jax.ShapeDtypeStruct((N, D), jnp.float32),
    )(x, w1, b1, w2, b2)


def _v2c_msg_body(x_ref, ef_ref, dg_ref, w1h, b1h, w2h, b2h, w1m, b1m, w2m, b2m, o_ref):
    x = x_ref[...]
    ef = ef_ref[...]
    deg = (dg_ref[0] + dg_ref[1])[:, 0:1]
    norm = jnp.where(deg > 0.5, 1.0 / deg, 0.0)
    t = jax.nn.sigmoid(x @ w1h[...] + b1h[...])
    va = (t @ w2h[...] + b2h[...])[:, 0:1] * ef
    u = jnp.maximum((ef * x) @ w1m[...] + b1m[...], 0.0)
    out = (u @ w2m[...] + b2m[...]) * norm
    o_ref[...] = jnp.where(_col_is_last(out.shape), va, out)


def _v2c_msg(x, ef, dg, wh, wm):
    return pl.pallas_call(
        _v2c_msg_body,
        grid=(GB,),
        in_specs=[_rows(D), _rows(1), _DEG_SPEC] +
                 [_full(w.shape) for w in wh] + [_full(w.shape) for w in wm],
        out_specs=_rows(D),
        out_shape=jax.ShapeDtypeStruct((N, D), jnp.float32),
    )(x, ef, dg, *wh, *wm)


def _v2c_upd_body(ag_ref, old_ref, rhs_ref, wr, br, o_ref):
    aggr = ag_ref[0] + ag_ref[1]
    main = jnp.maximum(aggr + old_ref[...] @ wr[...] + br[...], 0.0)
    last = aggr[:, D - 1:D] - rhs_ref[...]
    o_ref[...] = jnp.where(_col_is_last(main.shape), last, main)


def _v2c_upd(ag, old, rhs2, wr, br):
    return pl.pallas_call(
        _v2c_upd_body,
        grid=(GB,),
        in_specs=[_AGG_SPEC, _rows(D), _rows(1), _full(wr.shape), _full(br.shape)],
        out_specs=_rows(D),
        out_shape=jax.ShapeDtypeStruct((N, D), jnp.float32),
    )(ag, old, rhs2, wr, br)


def _c2v_msg_body(x_ref, ef_ref, dg_ref, w1m, b1m, w2m, b2m, o_ref):
    x = x_ref[...]
    ef = ef_ref[...]
    deg = (dg_ref[0] + dg_ref[1])[:, 0:1]
    norm = jnp.where(deg > 0.5, 1.0 / deg, 0.0)
    u = jnp.maximum((ef * x) @ w1m[...] + b1m[...], 0.0)
    out = u @ w2m[...] + b2m[...]
    bscal = x[:, D - 1:D] * ef
    o_ref[...] = norm * jnp.where(_col_is_last(out.shape), bscal, out)


def _c2v_msg(x, ef, dg, wm):
    return pl.pallas_call(
        _c2v_msg_body,
        grid=(GB,),
        in_specs=[_rows(D), _rows(1), _DEG_SPEC] + [_full(w.shape) for w in wm],
        out_specs=_rows(D),
        out_shape=jax.ShapeDtypeStruct((N, D), jnp.float32),
    )(x, ef, dg, *wm)


def _c2v_upd_body(ag_ref, xd_ref, w1h, b1h, w2h, b2h, wr, br, o_ref):
    aggr = ag_ref[0] + ag_ref[1]
    xd = xd_ref[...]
    t = jax.nn.sigmoid(xd @ w1h[...] + b1h[...])
    a = (t @ w2h[...] + b2h[...])[:, 0:1]
    main = jnp.maximum(aggr + xd @ wr[...] + br[...], 0.0)
    last = a * aggr[:, D - 1:D]
    o_ref[...] = jnp.where(_col_is_last(main.shape), last, main)


def _c2v_upd(ag, xd, wh, wr, br):
    return pl.pallas_call(
        _c2v_upd_body,
        grid=(GB,),
        in_specs=[_AGG_SPEC, _rows(D)] + [_full(w.shape) for w in wh] +
                 [_full(wr.shape), _full(br.shape)],
        out_specs=_rows(D),
        out_shape=jax.ShapeDtypeStruct((N, D), jnp.float32),
    )(ag, xd, *wh, wr, br)


def _embed2_body(v_ref, c_ref, vw1, vb1, vw2, vb2, cw1, cb1, cw2, cb2, ov, oc):
    hv = jnp.maximum(v_ref[...] @ vw1[...] + vb1[...], 0.0)
    ov[...] = hv @ vw2[...] + vb2[...]
    hc = jnp.maximum(c_ref[...] @ cw1[...] + cb1[...], 0.0)
    oc[...] = hc @ cw2[...] + cb2[...]


def _embed2(vnf, cnf, wv, wc):
    return pl.pallas_call(
        _embed2_body,
        grid=(GB,),
        in_specs=[_rows(2), _rows(2)] + [_full(w.shape) for w in wv] +
                 [_full(w.shape) for w in wc],
        out_specs=(_rows(D), _rows(D)),
        out_shape=(jax.ShapeDtypeStruct((N, D), jnp.float32),
                   jax.ShapeDtypeStruct((N, D), jnp.float32)),
    )(vnf, cnf, *wv, *wc)


def _upd_cmsg_body(ag_ref, old_ref, rhs_ref, wr, br, ef_ref, dg_ref,
                   w1m, b1m, w2m, b2m, ocons, omc):
    aggr = ag_ref[0] + ag_ref[1]
    main = jnp.maximum(aggr + old_ref[...] @ wr[...] + br[...], 0.0)
    last = aggr[:, D - 1:D] - rhs_ref[...]
    cons = jnp.where(_col_is_last(main.shape), last, main)
    ocons[...] = cons
    ef = ef_ref[...]
    deg = (dg_ref[0] + dg_ref[1])[:, 0:1]
    norm = jnp.where(deg > 0.5, 1.0 / deg, 0.0)
    u = jnp.maximum((ef * cons) @ w1m[...] + b1m[...], 0.0)
    out = u @ w2m[...] + b2m[...]
    bscal = cons[:, D - 1:D] * ef
    omc[...] = norm * jnp.where(_col_is_last(out.shape), bscal, out)


def _upd_cmsg(ag, old, rhs2, wr, br, ef, dg, wm):
    return pl.pallas_call(
        _upd_cmsg_body,
        grid=(GB,),
        in_specs=[_AGG_SPEC, _rows(D), _rows(1), _full(wr.shape),
                  _full(br.shape), _rows(1), _DEG_SPEC] +
                 [_full(w.shape) for w in wm],
        out_specs=(_rows(D), _rows(D)),
        out_shape=(jax.ShapeDtypeStruct((N, D), jnp.float32),
                   jax.ShapeDtypeStruct((N, D), jnp.float32)),
    )(ag, old, rhs2, wr, br, ef, dg, *wm)


def _cupd_vmsg_body(ag_ref, xd_ref, w1h, b1h, w2h, b2h, wr, br, ef_ref, dg_ref,
                    nw1h, nb1h, nw2h, nb2h, nw1m, nb1m, nw2m, nb2m, ovars, omv):
    aggr = ag_ref[0] + ag_ref[1]
    xd = xd_ref[...]
    t = jax.nn.sigmoid(xd @ w1h[...] + b1h[...])
    a = (t @ w2h[...] + b2h[...])[:, 0:1]
    main = jnp.maximum(aggr + xd @ wr[...] + br[...], 0.0)
    last = a * aggr[:, D - 1:D]
    vrs = jnp.where(_col_is_last(main.shape), last, main)
    ovars[...] = vrs
    ef = ef_ref[...]
    deg = (dg_ref[0] + dg_ref[1])[:, 0:1]
    norm = jnp.where(deg > 0.5, 1.0 / deg, 0.0)
    t2 = jax.nn.sigmoid(vrs @ nw1h[...] + nb1h[...])
    va = (t2 @ nw2h[...] + nb2h[...])[:, 0:1] * ef
    u = jnp.maximum((ef * vrs) @ nw1m[...] + nb1m[...], 0.0)
    out = (u @ nw2m[...] + nb2m[...]) * norm
    omv[...] = jnp.where(_col_is_last(out.shape), va, out)


def _cupd_vmsg(ag, xd, wh, wr, br, ef, dg, nwh, nwm):
    return pl.pallas_call(
        _cupd_vmsg_body,
        grid=(GB,),
        in_specs=[_AGG_SPEC, _rows(D)] + [_full(w.shape) for w in wh] +
                 [_full(wr.shape), _full(br.shape), _rows(1), _DEG_SPEC] +
                 [_full(w.shape) for w in nwh] + [_full(w.shape) for w in nwm],
        out_specs=(_rows(D), _rows(D)),
        out_shape=(jax.ShapeDtypeStruct((N, D), jnp.float32),
                   jax.ShapeDtypeStruct((N, D), jnp.float32)),
    )(ag, xd, *wh, wr, br, ef, dg, *nwh, *nwm)


def _cupd_head_body(ag_ref, xd_ref, w1h, b1h, w2h, b2h, wr, br, *refs):
    o_ref = refs[-1]
    ws = refs[:-1]
    aggr = ag_ref[0] + ag_ref[1]
    xd = xd_ref[...]
    t = jax.nn.sigmoid(xd @ w1h[...] + b1h[...])
    a = (t @ w2h[...] + b2h[...])[:, 0:1]
    main = jnp.maximum(aggr + xd @ wr[...] + br[...], 0.0)
    last = a * aggr[:, D - 1:D]
    x = jnp.where(_col_is_last(main.shape), last, main)
    for i in range(5):
        x = jnp.maximum(x @ ws[2 * i][...] + ws[2 * i + 1][...], 0.0)
    lg = x @ ws[10][...] + ws[11][...]
    o_ref[...] = jax.nn.log_softmax(lg, axis=1)


def _cupd_head(ag, xd, wh, wr, br, ws):
    return pl.pallas_call(
        _cupd_head_body,
        grid=(GB,),
        in_specs=[_AGG_SPEC, _rows(D)] + [_full(w.shape) for w in wh] +
                 [_full(wr.shape), _full(br.shape)] +
                 [_full(w.shape) for w in ws],
        out_specs=_rows(2),
        out_shape=jax.ShapeDtypeStruct((N, 2), jnp.float32),
    )(ag, xd, *wh, wr, br, *ws)


def _head_body(x_ref, *refs):
    o_ref = refs[-1]
    ws = refs[:-1]
    x = x_ref[...]
    for i in range(5):
        x = jnp.maximum(x @ ws[2 * i][...] + ws[2 * i + 1][...], 0.0)
    lg = x @ ws[10][...] + ws[11][...]
    o_ref[...] = jax.nn.log_softmax(lg, axis=1)


def _head(x, ws):
    return pl.pallas_call(
        _head_body,
        grid=(GB,),
        in_specs=[_rows(D)] + [_full(w.shape) for w in ws],
        out_specs=_rows(2),
        out_shape=jax.ShapeDtypeStruct((N, 2), jnp.float32),
    )(x, *ws)


def _pad_mlp2(p, pre):
    w1 = jnp.pad(p[pre + '_W1'], ((0, 0), (0, 1)))
    b1 = jnp.pad(p[pre + '_b1'], (0, 1)).reshape(1, D)
    w2 = jnp.pad(p[pre + '_W2'], ((0, 1), (0, 1)))
    b2 = jnp.pad(p[pre + '_b2'], (0, 1)).reshape(1, D)
    return (w1, b1, w2, b2)


def _pad_h2v(p, l):
    w1 = jnp.pad(p['h2v%d_W1' % l], ((0, 0), (0, 1)))
    b1 = jnp.pad(p['h2v%d_b1' % l], (0, 1)).reshape(1, D)
    w2 = jnp.pad(p['h2v%d_W2' % l], ((0, 1), (0, D - 1)))
    b2 = jnp.pad(p['h2v%d_b2' % l], (0, D - 1)).reshape(1, D)
    return (w1, b1, w2, b2)


def _pad_root(p, pre):
    wr = jnp.pad(p[pre + '_root'], ((0, 0), (0, 1)))
    br = jnp.pad(p[pre + '_bias'], (0, 1)).reshape(1, D)
    return wr, br


def _edge_layouts(src, dst):
    pad0 = jnp.zeros((PADN,), jnp.int32)
    pads = jnp.full((PADN,), SENT, jnp.int32)
    g_src = jnp.concatenate([src, pad0]).reshape(ROWS, KM)
    g_dst = jnp.concatenate([dst, pads]).reshape(ROWS, KM)
    d_src = jnp.concatenate([src, pads]).reshape(ROWS, KM)
    return g_src, g_dst, d_src


def _impl(vnf, cnf, eiv, efv, rhs, eic, efc, asums, params):
    del asums
    p = params
    sv = eiv[0].astype(jnp.int32)
    dv = eiv[1].astype(jnp.int32)
    sc = eic[0].astype(jnp.int32)
    dc = eic[1].astype(jnp.int32)
    svg, dvg, svd = _edge_layouts(sv, dv)
    scg, dcg, scd = _edge_layouts(sc, dc)
    zero_d = jnp.zeros((NPAD, D), jnp.float32)
    zero_w = jnp.zeros((NPAD, DW), jnp.float32)
    ones_w = jnp.ones((KM, DW), jnp.float32)
    efv = efv.astype(jnp.float32)
    efc = efc.astype(jnp.float32)
    rhs2 = rhs.astype(jnp.float32).reshape(N, 1)

    deg_v = _deg(svd, ones_w, zero_w)
    deg_c = _deg(scd, ones_w, zero_w)

    v0, c0 = _embed2(vnf, cnf, _pad_mlp2(p, 'con_mlp'), _pad_mlp2(p, 'var_mlp'))

    ws = []
    for i in range(1, 6):
        ws += [p['fc%d_W' % i], p['fc%d_b' % i].reshape(1, D)]
    ws += [p['fc6_W'], p['fc6_b'].reshape(1, 2)]

    mv = _v2c_msg(v0, efv, deg_v, _pad_h2v(p, 1), _pad_mlp2(p, 'v2c1'))
    old_cons, vars_ = c0, v0
    for l in (1, 2, 3, 4):
        wh = _pad_h2v(p, l)
        ag = _spmm(mv, svg, dvg, zero_d)
        cons, mc = _upd_cmsg(ag, old_cons, rhs2, *_pad_root(p, 'v2c%d' % l),
                             efc, deg_c, _pad_mlp2(p, 'c2v%d' % l))
        ag2 = _spmm(mc, scg, dcg, zero_d)
        wr, br = _pad_root(p, 'c2v%d' % l)
        if l < 4:
            vars_, mv = _cupd_vmsg(ag2, vars_, wh, wr, br, efv, deg_v,
                                   _pad_h2v(p, l + 1),
                                   _pad_mlp2(p, 'v2c%d' % (l + 1)))
            old_cons = cons
        else:
            return _cupd_head(ag2, vars_, wh, wr, br, ws)


def kernel(var_node_features, con_node_features, edge_index_var, edge_features_var, rhs, edge_index_con, edge_features_con, asums, params):
    return _impl(var_node_features, con_node_features, edge_index_var,
                 edge_features_var, rhs, edge_index_con, edge_features_con,
                 asums, params)
